# baseline (speedup 1.0000x reference)
import jax
import jax.numpy as jnp
from jax import lax
from jax.experimental import pallas as pl
from jax.experimental.pallas import tpu as pltpu
from jax.experimental.pallas import tpu_sc as plsc

N = 10000
E = 160000
H = 64
IN = 16

NC, NS, L = 2, 16, 16
NW = NC * NS
CH = 128
EPW = E // NW
NCHUNK = (EPW + CH - 1) // CH
EPW_PAD = NCHUNK * CH
E_PAD = EPW_PAD * NW
NCHT = NCHUNK + 2
E_TAB = NCHT * CH * NW
N_PAD = 10240
RPT = N_PAD // NS
RB = 2000
NRB = N // RB

_MESH = plsc.VectorSubcoreMesh(core_axis_name="c", subcore_axis_name="s")


def _make_prop(d, nh):
  nseg = d // L
  nbuf = 2 if d > 80 or nh == 2 else 3
  assert NCHT % nbuf == 0

  scratch = [
      pltpu.VMEM((NCHT, CH), jnp.int32),
      pltpu.VMEM((NCHT, CH), jnp.int32),
      pltpu.VMEM_SHARED((N_PAD, d), jnp.float32),
  ]
  scratch += [pltpu.VMEM((CH, d), jnp.float32) for _ in range(nbuf)]
  scratch += [pltpu.VMEM((CH, L), jnp.float32) for _ in range(nbuf)]
  scratch += [pltpu.SemaphoreType.DMA for _ in range(3 * nbuf)]
  if nh == 2:
    scratch += [pltpu.VMEM((NCHT, CH), jnp.int32)]
    scratch += [pltpu.VMEM((CH, d), jnp.float32) for _ in range(nbuf)]
    scratch += [pltpu.SemaphoreType.DMA for _ in range(nbuf)]

  def body(y, src_h, dst_h, w_h, zeros_h, out, *rest):
    srcv, dstv, acc = rest[0], rest[1], rest[2]
    rows = rest[3:3 + nbuf]
    wv = rest[3 + nbuf:3 + 2 * nbuf]
    semg = rest[3 + 2 * nbuf:3 + 3 * nbuf]
    sems = rest[3 + 3 * nbuf:3 + 4 * nbuf]
    semw = rest[3 + 4 * nbuf:3 + 5 * nbuf]
    if nh == 2:
      src2v = rest[3 + 5 * nbuf]
      rows2 = rest[4 + 5 * nbuf:4 + 6 * nbuf]
      semg2 = rest[4 + 6 * nbuf:4 + 7 * nbuf]
    c = lax.axis_index("c")
    s = lax.axis_index("s")
    wid = c * NS + s
    pltpu.sync_copy(src_h.at[wid], srcv)
    pltpu.sync_copy(dst_h.at[wid], dstv)
    if nh == 2:
      pltpu.sync_copy(src_h.at[NW + wid], src2v)
    r0 = s * RPT
    pltpu.sync_copy(zeros_h.at[pl.ds(r0, RPT)], acc.at[pl.ds(r0, RPT)])
    plsc.subcore_barrier()

    def start_g(j, b):
      pltpu.async_copy(y.at[srcv.at[j]], rows[b], semg[b])
      pltpu.async_copy(w_h.at[wid, j], wv[b], semw[b])
      if nh == 2:
        pltpu.async_copy(y.at[src2v.at[j]], rows2[b], semg2[b])

    def wait_g(b):
      pltpu.make_async_copy(y.at[srcv.at[0]], rows[b], semg[b]).wait()
      pltpu.make_async_copy(w_h.at[wid, 0], wv[b], semw[b]).wait()
      if nh == 2:
        pltpu.make_async_copy(y.at[src2v.at[0]], rows2[b], semg2[b]).wait()

    def wait_s(b):
      pltpu.make_async_copy(rows[b], acc.at[dstv.at[0]], sems[b]).wait()

    def scale(b):
      rb = rows[b]
      r2b = rows2[b] if nh == 2 else None
      wb = wv[b]

      @plsc.parallel_loop(0, CH, unroll=8)
      def _(e):
        we = wb[e, pl.ds(0, L)]
        for k in range(nseg):
          seg = rb[e, pl.ds(k * L, L)]
          if nh == 2:
            seg = seg + r2b[e, pl.ds(k * L, L)]
          rb[e, pl.ds(k * L, L)] = seg * we

    def prefetch(j, b):
      nxt = j + nbuf - 1
      if isinstance(nxt, int):
        if nxt < NCHT:
          start_g(nxt, (b + nbuf - 1) % nbuf)
      else:
        @pl.when(nxt < NCHT)
        def _():
          start_g(nxt, (b + nbuf - 1) % nbuf)

    def phase(j, b, first=False, last=False):
      wait_g(b)
      if nbuf == 2:
        if not first:
          wait_s((b + 1) % 2)
        if not last:
          prefetch(j, b)
        scale(b)
        pltpu.async_copy(rows[b], acc.at[dstv.at[j]], sems[b], add=True)
      else:
        scale(b)
        pltpu.async_copy(rows[b], acc.at[dstv.at[j]], sems[b], add=True)
        if not first:
          wait_s((b + 2) % 3)
        if not last:
          prefetch(j, b)

    for j0 in range(nbuf - 1):
      start_g(j0, j0)
    phase(0, 0, first=True)
    for j0 in range(1, nbuf - 1):
      phase(j0, j0)

    def outer(t, carry):
      jbase = (nbuf - 1) + nbuf * t
      for p in range(nbuf):
        phase(jbase + p, (nbuf - 1 + p) % nbuf)
      return carry

    lax.fori_loop(0, (NCHT - nbuf) // nbuf, outer, 0)
    phase(NCHT - 1, (NCHT - 1) % nbuf, last=True)
    wait_s((NCHT - 1) % nbuf)
    plsc.subcore_barrier()
    pltpu.sync_copy(acc.at[pl.ds(r0, RPT)], out.at[c, pl.ds(r0, RPT)])

  return pl.kernel(
      body,
      out_type=jax.ShapeDtypeStruct((2, N_PAD, d), jnp.float32),
      mesh=_MESH,
      scratch_types=scratch,
      compiler_params=pltpu.CompilerParams(use_tc_tiling_on_sc=False),
  )


def _make_prop2(d1, d2, out16):
  nseg1, nseg2 = d1 // L, d2 // L
  nbuf = 2
  same = (d1 == d2) and not out16

  scratch = [
      pltpu.VMEM((NCHT, CH), jnp.int32),
      pltpu.VMEM((NCHT, CH), jnp.int32),
      pltpu.VMEM_SHARED((N_PAD, d1), jnp.float32),
      pltpu.VMEM_SHARED((N_PAD, d2), jnp.float32),
  ]
  scratch += [pltpu.VMEM((CH, d1), jnp.float32) for _ in range(nbuf)]
  if not same:
    scratch += [pltpu.VMEM((CH, d2), jnp.float32) for _ in range(nbuf)]
  scratch += [pltpu.VMEM((CH, L), jnp.float32) for _ in range(nbuf)]
  scratch += [pltpu.SemaphoreType.DMA for _ in range(3 * nbuf)]

  outs = [jax.ShapeDtypeStruct((2 * N_PAD, d1), jnp.float32),
          jax.ShapeDtypeStruct((2, N_PAD, d2), jnp.float32)]
  if out16:
    outs.append(jax.ShapeDtypeStruct((2 * N_PAD, 16), jnp.float32))

  def body(y, src2_h, dst_h, w_h, zA, zB, *rest):
    outA, outB = rest[0], rest[1]
    rest = rest[2:]
    if out16:
      outA16 = rest[0]
      rest = rest[1:]
    srcv, dstv, accA, accB = rest[0], rest[1], rest[2], rest[3]
    rest = rest[4:]
    rowsA = rest[:nbuf]
    rest = rest[nbuf:]
    if same:
      rowsB = rowsA
    else:
      rowsB = rest[:nbuf]
      rest = rest[nbuf:]
    wv = rest[:nbuf]
    semg = rest[nbuf:2 * nbuf]
    sems = rest[2 * nbuf:3 * nbuf]
    semw = rest[3 * nbuf:4 * nbuf]
    c = lax.axis_index("c")
    s = lax.axis_index("s")
    widA = c * NS + s
    widB = (1 - c) * NS + s
    r0 = s * RPT
    pltpu.sync_copy(zA.at[pl.ds(r0, RPT)], accA.at[pl.ds(r0, RPT)])
    pltpu.sync_copy(zB.at[pl.ds(r0, RPT)], accB.at[pl.ds(r0, RPT)])

    def run_edges(ysrc, rows, nseg, src_row, widw, acc):
      pltpu.sync_copy(src2_h.at[src_row], srcv)
      pltpu.sync_copy(dst_h.at[widw], dstv)

      def start_g(j, b):
        pltpu.async_copy(ysrc.at[srcv.at[j]], rows[b], semg[b])
        pltpu.async_copy(w_h.at[widw, j], wv[b], semw[b])

      def wait_g(b):
        pltpu.make_async_copy(ysrc.at[srcv.at[0]], rows[b], semg[b]).wait()
        pltpu.make_async_copy(w_h.at[widw, 0], wv[b], semw[b]).wait()

      def wait_s(b):
        pltpu.make_async_copy(rows[b], acc.at[dstv.at[0]], sems[b]).wait()

      def scale(b):
        rb = rows[b]
        wb = wv[b]

        @plsc.parallel_loop(0, CH, unroll=8)
        def _(e):
          we = wb[e, pl.ds(0, L)]
          for k in range(nseg):
            rb[e, pl.ds(k * L, L)] = rb[e, pl.ds(k * L, L)] * we

      def phase(j, b, first=False, last=False):
        wait_g(b)
        if not first:
          wait_s((b + 1) % 2)
        if not last:
          nxt = j + 1
          if isinstance(nxt, int):
            if nxt < NCHT:
              start_g(nxt, (b + 1) % 2)
          else:
            @pl.when(nxt < NCHT)
            def _():
              start_g(nxt, (b + 1) % 2)
        scale(b)
        pltpu.async_copy(rows[b], acc.at[dstv.at[j]], sems[b], add=True)

      start_g(0, 0)
      phase(0, 0, first=True)

      def outer(t, carry):
        jbase = 1 + 2 * t
        phase(jbase, 1)
        phase(jbase + 1, 0)
        return carry

      lax.fori_loop(0, (NCHT - 2) // 2, outer, 0)
      phase(NCHT - 1, (NCHT - 1) % 2, last=True)
      wait_s((NCHT - 1) % 2)

    plsc.subcore_barrier()
    run_edges(y, rowsA, nseg1, widA, widA, accA)
    plsc.subcore_barrier()
    pltpu.sync_copy(accA.at[pl.ds(r0, RPT)],
                    outA.at[pl.ds(c * N_PAD + r0, RPT)])
    if out16:
      pltpu.sync_copy(accA.at[pl.ds(r0, RPT), pl.ds(0, 16)],
                      outA16.at[pl.ds(c * N_PAD + r0, RPT)])
    plsc.subcore_barrier()
    yB = outA16 if out16 else outA
    run_edges(yB, rowsB, nseg2, c * NW + widA, widA, accB)
    run_edges(yB, rowsB, nseg2, c * NW + widB, widB, accB)
    plsc.subcore_barrier()
    pltpu.sync_copy(accB.at[pl.ds(r0, RPT)], outB.at[c, pl.ds(r0, RPT)])

  return pl.kernel(
      body,
      out_type=tuple(outs),
      mesh=_MESH,
      scratch_types=scratch,
      compiler_params=pltpu.CompilerParams(use_tc_tiling_on_sc=False),
  )


_PROP2S = {}


def _prop2(d1, d2, out16):
  key = (d1, d2, out16)
  if key not in _PROP2S:
    _PROP2S[key] = _make_prop2(d1, d2, out16)
  return _PROP2S[key]


def _make_norm():
  scratch = [
      pltpu.VMEM((NCHT, CH), jnp.int32),
      pltpu.VMEM((NCHT, CH), jnp.float32),
      pltpu.VMEM((CH,), jnp.float32),
      pltpu.SemaphoreType.DMA,
  ]

  def body(recip_h, dst_h, ew_h, wout, dstv, wv, rbuf, sem):
    c = lax.axis_index("c")
    s = lax.axis_index("s")
    wid = c * NS + s
    pltpu.sync_copy(dst_h.at[wid], dstv)
    pltpu.sync_copy(ew_h.at[wid], wv)

    def chunk(j, carry):
      pltpu.async_copy(recip_h.at[dstv.at[j]], rbuf, sem).wait()
      for k in range(CH // L):
        wv[j, pl.ds(k * L, L)] = wv[j, pl.ds(k * L, L)] * rbuf[pl.ds(k * L, L)]
      return carry

    lax.fori_loop(0, NCHT, chunk, 0)
    pltpu.sync_copy(wv, wout.at[wid])

  return pl.kernel(
      body,
      out_type=jax.ShapeDtypeStruct((NW, NCHT, CH), jnp.float32),
      mesh=_MESH,
      scratch_types=scratch,
      compiler_params=pltpu.CompilerParams(use_tc_tiling_on_sc=False),
  )


_PROPS = {}


def _prop(d, nh):
  key = (d, nh)
  if key not in _PROPS:
    _PROPS[key] = _make_prop(d, nh)
  return _PROPS[key]


_NORM = _make_norm()


def _dot(a, b):
  return jax.lax.dot_general(a, b, (((1,), (0,)), ((), ())),
                             preferred_element_type=jnp.float32)


def _tc_recip_body(deg_h, recip_o):
  deg = deg_h[0, 0:N, 0:1] + deg_h[1, 0:N, 0:1]
  recip_o[...] = (1.0 / jnp.maximum(deg, 1e-6))[:, 0]


def _tc_recip(deg_h):
  return pl.pallas_call(
      _tc_recip_body,
      out_shape=jax.ShapeDtypeStruct((N,), jnp.float32),
  )(deg_h)


def _row(d):
  return pl.BlockSpec((RB, d), lambda i: (i, 0))


def _full(*shape):
  return pl.BlockSpec(shape, lambda i: tuple(0 for _ in shape))


def _chalf(d):
  return pl.BlockSpec((2, RB, d), lambda i: (0, i, 0))


def _tc_a_body(h, Wf, bf, Wi0, bi, xs1_o, y1_o):
  xs1 = _dot(h[...], Wf[...]) + bf[...]
  z = _dot(xs1, Wi0[...]) + bi[...]
  xs1_o[...] = xs1
  y1_o[...] = jnp.concatenate([z, h[...]], axis=-1)


def _tc_a(h, Wf, bf, Wi0, bi):
  return pl.pallas_call(
      _tc_a_body,
      grid=(NRB,),
      in_specs=[_row(H), _full(H, H), _full(1, H), _full(H, H), _full(1, H)],
      out_specs=(_row(H), _row(2 * H)),
      out_shape=(jax.ShapeDtypeStruct((N, H), jnp.float32),
                 jax.ShapeDtypeStruct((N, 2 * H), jnp.float32)),
  )(h, Wf, bf, Wi0, bi)


def _tc_b_body(C1, h, Wg, bg, Wo0, Wo1, bo, alpha, Wro0, Wro1, bro,
               rep_o, xs2_o, y2_o):
  Az = C1[0, :, 0:H] + C1[1, :, 0:H]
  Ah = C1[0, :, H:2 * H] + C1[1, :, H:2 * H]
  conv = _dot(Az, Wg[...]) + bg[...]
  o1 = _dot(conv, Wo0[...]) + _dot(h[...], Wo1[...]) + bo[...]
  out = jnp.where(o1 > 0, o1, alpha[0, 0] * o1)
  rep_o[...] = jnp.concatenate([out, h[...]], axis=-1)
  xs2 = _dot(out, Wro0[...]) + _dot(h[...], Wro1[...]) + bro[...]
  xs2_o[...] = xs2
  y2_o[...] = jnp.concatenate([xs2, Ah], axis=-1)


def _tc_b(C1, h, Wg, bg, Wo0, Wo1, bo, alpha, Wro0, Wro1, bro):
  return pl.pallas_call(
      _tc_b_body,
      grid=(NRB,),
      in_specs=[_chalf(2 * H), _row(H), _full(H, H), _full(1, H),
                _full(H, H), _full(H, H), _full(1, H), _full(1, 1),
                _full(H, IN), _full(H, IN), _full(1, IN)],
      out_specs=(_row(2 * H), _row(IN), _row(IN + H)),
      out_shape=(jax.ShapeDtypeStruct((N, 2 * H), jnp.float32),
                 jax.ShapeDtypeStruct((N, IN), jnp.float32),
                 jax.ShapeDtypeStruct((N, IN + H), jnp.float32)),
  )(C1, h, Wg, bg, Wo0, Wo1, bo, alpha, Wro0, Wro1, bro)


def _tc_c_body(C2, xs2, h, y2,
               Wr0a, Wr0b, Wr1a, Wr1b, Wr2b, br,
               Wu0a, Wu0b, Wu1a, Wu1b, Wu2b, bu,
               axs2_o, prer_o, preu_o):
  Axs2 = C2[0, :, 0:IN] + C2[1, :, 0:IN]
  A2h = C2[0, :, IN:IN + H] + C2[1, :, IN:IN + H]
  Ah = y2[:, IN:IN + H]
  xs2v = xs2[...]
  hv = h[...]
  prer = (_dot(xs2v, Wr0a[...]) + _dot(hv, Wr0b[...]) + _dot(Axs2, Wr1a[...])
          + _dot(Ah, Wr1b[...]) + _dot(A2h, Wr2b[...]) + br[...])
  preu = (_dot(xs2v, Wu0a[...]) + _dot(hv, Wu0b[...]) + _dot(Axs2, Wu1a[...])
          + _dot(Ah, Wu1b[...]) + _dot(A2h, Wu2b[...]) + bu[...])
  axs2_o[...] = Axs2
  prer_o[...] = prer
  preu_o[...] = preu


def _tc_c(C2, xs2, h, y2, wr, wu):
  wspecs = [_full(IN, H), _full(H, H), _full(IN, H), _full(H, H),
            _full(H, H), _full(1, H)]
  return pl.pallas_call(
      _tc_c_body,
      grid=(NRB,),
      in_specs=[_chalf(IN + H), _row(IN), _row(H), _row(IN + H)]
               + wspecs + wspecs,
      out_specs=(_row(IN), _row(H), _row(H)),
      out_shape=(jax.ShapeDtypeStruct((N, IN), jnp.float32),
                 jax.ShapeDtypeStruct((N, H), jnp.float32),
                 jax.ShapeDtypeStruct((N, H), jnp.float32)),
  )(C2, xs2, h, y2, *wr, *wu)


def _tc_d_body(C3, prer, preu, h, xs2, Axs2,
               Wr2a, Wu2a, Wc0a, Wc0b, Wc1a, Wc2a, bc,
               rh_o, prec_o, u_o):
  A2xs2 = C3[0] + C3[1]
  r = jax.nn.sigmoid(prer[...] + _dot(A2xs2, Wr2a[...]))
  u = jax.nn.sigmoid(preu[...] + _dot(A2xs2, Wu2a[...]))
  rh = r * h[...]
  prec = (_dot(xs2[...], Wc0a[...]) + _dot(rh, Wc0b[...])
          + _dot(Axs2[...], Wc1a[...]) + _dot(A2xs2, Wc2a[...]) + bc[...])
  rh_o[...] = rh
  prec_o[...] = prec
  u_o[...] = u


def _tc_d(C3, prer, preu, h, xs2, Axs2, Wr2a, Wu2a, Wc0a, Wc0b, Wc1a, Wc2a, bc):
  return pl.pallas_call(
      _tc_d_body,
      grid=(NRB,),
      in_specs=[_chalf(IN), _row(H), _row(H), _row(H), _row(IN), _row(IN),
                _full(IN, H), _full(IN, H), _full(IN, H), _full(H, H),
                _full(IN, H), _full(IN, H), _full(1, H)],
      out_specs=(_row(H), _row(H), _row(H)),
      out_shape=(jax.ShapeDtypeStruct((N, H), jnp.float32),
                 jax.ShapeDtypeStruct((N, H), jnp.float32),
                 jax.ShapeDtypeStruct((N, H), jnp.float32)),
  )(C3, prer, preu, h, xs2, Axs2, Wr2a, Wu2a, Wc0a, Wc0b, Wc1a, Wc2a, bc)


def _tc_f_body(C4, C5, prec, u, h, Wc1b, Wc2b, hnew_o):
  Arh = C4[0] + C4[1]
  A2rh = C5[0] + C5[1]
  c = jnp.tanh(prec[...] + _dot(Arh, Wc1b[...]) + _dot(A2rh, Wc2b[...]))
  uv = u[...]
  hnew_o[...] = uv * h[...] + (1.0 - uv) * c


def _tc_f(C4, C5, prec, u, h, Wc1b, Wc2b):
  return pl.pallas_call(
      _tc_f_body,
      grid=(NRB,),
      in_specs=[_chalf(H), _chalf(H), _row(H), _row(H), _row(H),
                _full(H, H), _full(H, H)],
      out_specs=_row(H),
      out_shape=jax.ShapeDtypeStruct((N, H), jnp.float32),
  )(C4, C5, prec, u, h, Wc1b, Wc2b)


def kernel(x, edge_index, edge_weight, Wr, br, Wu, bu, Wc, bc, Wf, bf,
           Wi, bi, Wg, bg, Wo, bo, alpha, Wro, bro):
  S = x.shape[1]

  src = edge_index[0].astype(jnp.int32)
  dst = edge_index[1].astype(jnp.int32)
  npad = E_PAD - E
  pad_idx = (jnp.arange(npad, dtype=jnp.int32) * 37) % N
  extra = ((jnp.arange(NW * 2 * CH, dtype=jnp.int32) * 37) % N
           ).reshape(NW, 2, CH)
  src_p = jnp.concatenate([
      jnp.concatenate([src, pad_idx]).reshape(NW, NCHUNK, CH), extra], axis=1)
  dst_p = jnp.concatenate([
      jnp.concatenate([dst, pad_idx]).reshape(NW, NCHUNK, CH), extra], axis=1)
  ew_p = jnp.concatenate([
      jnp.concatenate(
          [edge_weight.astype(jnp.float32), jnp.zeros((npad,), jnp.float32)]
      ).reshape(NW, NCHUNK, CH),
      jnp.zeros((NW, 2, CH), jnp.float32)], axis=1)
  src2_p = jnp.concatenate([src_p, src_p + N_PAD], axis=0)

  z16 = jnp.zeros((N_PAD, 16), jnp.float32)
  z64 = jnp.zeros((N_PAD, 64), jnp.float32)
  z80 = jnp.zeros((N_PAD, 80), jnp.float32)
  z128 = jnp.zeros((N_PAD, 128), jnp.float32)
  ones16 = jnp.ones((N, 16), jnp.float32)

  _rep = lambda t: jnp.broadcast_to(
      t.reshape(E_TAB, 1), (E_TAB, L)).reshape(NW, NCHT, CH, L)
  ew_rep = _rep(ew_p)
  deg_h = _prop(16, 1)(ones16, src_p, dst_p, ew_rep, z16)
  recip = _tc_recip(deg_h)
  wT = _NORM(recip, dst_p, ew_p)
  w_rep = _rep(wT)

  b2 = lambda b: b.reshape(1, -1)
  Wi0 = Wi[0:H]
  Wo0, Wo1 = Wo[0:H], Wo[H:2 * H]
  Wro0, Wro1 = Wro[0:H], Wro[H:2 * H]
  wr = (Wr[0:IN], Wr[32:96], Wr[96:112], Wr[128:192], Wr[224:288], b2(br))
  wu = (Wu[0:IN], Wu[32:96], Wu[96:112], Wu[128:192], Wu[224:288], b2(bu))
  Wr2a, Wu2a = Wr[192:208], Wu[192:208]
  Wc0a, Wc0b, Wc1a, Wc2a = Wc[0:IN], Wc[32:96], Wc[96:112], Wc[192:208]
  Wc1b, Wc2b = Wc[128:192], Wc[224:288]
  alpha2 = alpha.reshape(1, 1)

  h = jnp.zeros((N, H), jnp.float32)
  gens, preds, reprs, states = [], [], [], []
  for _ in range(S):
    xs1, y1 = _tc_a(h, Wf, b2(bf), Wi0, b2(bi))
    C1 = _prop(128, 1)(y1, src_p, dst_p, w_rep, z128)
    rep, xs2, y2 = _tc_b(C1, h, Wg, b2(bg), Wo0, Wo1, b2(bo), alpha2,
                         Wro0, Wro1, b2(bro))
    C2f, C3, _ = _prop2(80, 16, True)(y2, src2_p, dst_p, w_rep, z80, z16)
    C2 = C2f.reshape(2, N_PAD, 80)
    Axs2, prer, preu = _tc_c(C2, xs2, h, y2, wr, wu)
    rh, prec, u = _tc_d(C3, prer, preu, h, xs2, Axs2,
                        Wr2a, Wu2a, Wc0a, Wc0b, Wc1a, Wc2a, b2(bc))
    C4f, C5 = _prop2(64, 64, False)(rh, src2_p, dst_p, w_rep, z64, z64)
    C4 = C4f.reshape(2, N_PAD, H)
    h = _tc_f(C4, C5, prec, u, h, Wc1b, Wc2b)
    gens.append(xs2)
    preds.append(xs1)
    reprs.append(rep)
    states.append(h)

  generations = jnp.stack(gens, 0)[None]
  predictions = jnp.stack(preds, 0)[None]
  representations = jnp.stack(reprs, 0)[None]
  states_out = jnp.stack(states, 0)[None, None]
  return generations, predictions, representations, states_out


# speedup vs baseline: 1.0700x; 1.0700x over previous
import jax
import jax.numpy as jnp
from jax import lax
from jax.experimental import pallas as pl
from jax.experimental.pallas import tpu as pltpu
from jax.experimental.pallas import tpu_sc as plsc

N = 10000
E = 160000
H = 64
IN = 16

NC, NS, L = 2, 16, 16
NW = NC * NS
CH = 128
EPW = E // NW
NCHUNK = (EPW + CH - 1) // CH
EPW_PAD = NCHUNK * CH
E_PAD = EPW_PAD * NW
NCHT = NCHUNK + 2
E_TAB = NCHT * CH * NW
N_PAD = 10240
RPT = N_PAD // NS
RB = 2000
NRB = N // RB

_MESH = plsc.VectorSubcoreMesh(core_axis_name="c", subcore_axis_name="s")


def _make_prop(d, nh):
  nseg = d // L
  nbuf = 2 if d > 80 or nh == 2 else 3
  assert NCHT % nbuf == 0

  scratch = [
      pltpu.VMEM((NCHT, CH), jnp.int32),
      pltpu.VMEM((NCHT, CH), jnp.int32),
      pltpu.VMEM_SHARED((N_PAD, d), jnp.float32),
  ]
  scratch += [pltpu.VMEM((CH, d), jnp.float32) for _ in range(nbuf)]
  scratch += [pltpu.VMEM((CH, L), jnp.float32) for _ in range(nbuf)]
  scratch += [pltpu.SemaphoreType.DMA for _ in range(3 * nbuf)]
  if nh == 2:
    scratch += [pltpu.VMEM((NCHT, CH), jnp.int32)]
    scratch += [pltpu.VMEM((CH, d), jnp.float32) for _ in range(nbuf)]
    scratch += [pltpu.SemaphoreType.DMA for _ in range(nbuf)]

  def body(y, src_h, dst_h, w_h, zeros_h, out, *rest):
    srcv, dstv, acc = rest[0], rest[1], rest[2]
    rows = rest[3:3 + nbuf]
    wv = rest[3 + nbuf:3 + 2 * nbuf]
    semg = rest[3 + 2 * nbuf:3 + 3 * nbuf]
    sems = rest[3 + 3 * nbuf:3 + 4 * nbuf]
    semw = rest[3 + 4 * nbuf:3 + 5 * nbuf]
    if nh == 2:
      src2v = rest[3 + 5 * nbuf]
      rows2 = rest[4 + 5 * nbuf:4 + 6 * nbuf]
      semg2 = rest[4 + 6 * nbuf:4 + 7 * nbuf]
    c = lax.axis_index("c")
    s = lax.axis_index("s")
    wid = c * NS + s
    pltpu.sync_copy(src_h.at[wid], srcv)
    pltpu.sync_copy(dst_h.at[wid], dstv)
    if nh == 2:
      pltpu.sync_copy(src_h.at[NW + wid], src2v)
    r0 = s * RPT
    pltpu.sync_copy(zeros_h.at[pl.ds(r0, RPT)], acc.at[pl.ds(r0, RPT)])
    plsc.subcore_barrier()

    def start_g(j, b):
      pltpu.async_copy(y.at[srcv.at[j]], rows[b], semg[b])
      pltpu.async_copy(w_h.at[wid, j], wv[b], semw[b])
      if nh == 2:
        pltpu.async_copy(y.at[src2v.at[j]], rows2[b], semg2[b])

    def wait_g(b):
      pltpu.make_async_copy(y.at[srcv.at[0]], rows[b], semg[b]).wait()
      pltpu.make_async_copy(w_h.at[wid, 0], wv[b], semw[b]).wait()
      if nh == 2:
        pltpu.make_async_copy(y.at[src2v.at[0]], rows2[b], semg2[b]).wait()

    def wait_s(b):
      pltpu.make_async_copy(rows[b], acc.at[dstv.at[0]], sems[b]).wait()

    def scale(b):
      rb = rows[b]
      r2b = rows2[b] if nh == 2 else None
      wb = wv[b]

      @plsc.parallel_loop(0, CH, unroll=8)
      def _(e):
        we = wb[e, pl.ds(0, L)]
        for k in range(nseg):
          seg = rb[e, pl.ds(k * L, L)]
          if nh == 2:
            seg = seg + r2b[e, pl.ds(k * L, L)]
          rb[e, pl.ds(k * L, L)] = seg * we

    def prefetch(j, b):
      nxt = j + nbuf - 1
      if isinstance(nxt, int):
        if nxt < NCHT:
          start_g(nxt, (b + nbuf - 1) % nbuf)
      else:
        @pl.when(nxt < NCHT)
        def _():
          start_g(nxt, (b + nbuf - 1) % nbuf)

    def phase(j, b, first=False, last=False):
      wait_g(b)
      if nbuf == 2:
        if not first:
          wait_s((b + 1) % 2)
        if not last:
          prefetch(j, b)
        scale(b)
        pltpu.async_copy(rows[b], acc.at[dstv.at[j]], sems[b], add=True)
      else:
        scale(b)
        pltpu.async_copy(rows[b], acc.at[dstv.at[j]], sems[b], add=True)
        if not first:
          wait_s((b + 2) % 3)
        if not last:
          prefetch(j, b)

    for j0 in range(nbuf - 1):
      start_g(j0, j0)
    phase(0, 0, first=True)
    for j0 in range(1, nbuf - 1):
      phase(j0, j0)

    def outer(t, carry):
      jbase = (nbuf - 1) + nbuf * t
      for p in range(nbuf):
        phase(jbase + p, (nbuf - 1 + p) % nbuf)
      return carry

    lax.fori_loop(0, (NCHT - nbuf) // nbuf, outer, 0)
    phase(NCHT - 1, (NCHT - 1) % nbuf, last=True)
    wait_s((NCHT - 1) % nbuf)
    plsc.subcore_barrier()
    pltpu.sync_copy(acc.at[pl.ds(r0, RPT)], out.at[c, pl.ds(r0, RPT)])

  return pl.kernel(
      body,
      out_type=jax.ShapeDtypeStruct((2, N_PAD, d), jnp.float32),
      mesh=_MESH,
      scratch_types=scratch,
      compiler_params=pltpu.CompilerParams(use_tc_tiling_on_sc=False),
  )


def _make_prop2(d1, d2, out16):
  nseg1, nseg2 = d1 // L, d2 // L
  nbuf = 2
  same = (d1 == d2) and not out16

  scratch = [
      pltpu.VMEM((NCHT, CH), jnp.int32),
      pltpu.VMEM((NCHT, CH), jnp.int32),
      pltpu.VMEM_SHARED((N_PAD, d1), jnp.float32),
      pltpu.VMEM_SHARED((N_PAD, d2), jnp.float32),
  ]
  scratch += [pltpu.VMEM((CH, d1), jnp.float32) for _ in range(nbuf)]
  if not same:
    scratch += [pltpu.VMEM((CH, d2), jnp.float32) for _ in range(nbuf)]
  scratch += [pltpu.VMEM((CH, L), jnp.float32) for _ in range(nbuf)]
  scratch += [pltpu.SemaphoreType.DMA for _ in range(3 * nbuf)]

  outs = [jax.ShapeDtypeStruct((2 * N_PAD, d1), jnp.float32),
          jax.ShapeDtypeStruct((2, N_PAD, d2), jnp.float32)]
  if out16:
    outs.append(jax.ShapeDtypeStruct((2 * N_PAD, 16), jnp.float32))

  def body(y, src2_h, dst_h, w_h, zA, zB, *rest):
    outA, outB = rest[0], rest[1]
    rest = rest[2:]
    if out16:
      outA16 = rest[0]
      rest = rest[1:]
    srcv, dstv, accA, accB = rest[0], rest[1], rest[2], rest[3]
    rest = rest[4:]
    rowsA = rest[:nbuf]
    rest = rest[nbuf:]
    if same:
      rowsB = rowsA
    else:
      rowsB = rest[:nbuf]
      rest = rest[nbuf:]
    wv = rest[:nbuf]
    semg = rest[nbuf:2 * nbuf]
    sems = rest[2 * nbuf:3 * nbuf]
    semw = rest[3 * nbuf:4 * nbuf]
    c = lax.axis_index("c")
    s = lax.axis_index("s")
    widA = c * NS + s
    widB = (1 - c) * NS + s
    r0 = s * RPT
    pltpu.sync_copy(zA.at[pl.ds(r0, RPT)], accA.at[pl.ds(r0, RPT)])
    pltpu.sync_copy(zB.at[pl.ds(r0, RPT)], accB.at[pl.ds(r0, RPT)])

    def run_edges(ysrc, rows, nseg, src_row, widw, acc):
      pltpu.sync_copy(src2_h.at[src_row], srcv)
      pltpu.sync_copy(dst_h.at[widw], dstv)

      def start_g(j, b):
        pltpu.async_copy(ysrc.at[srcv.at[j]], rows[b], semg[b])
        pltpu.async_copy(w_h.at[widw, j], wv[b], semw[b])

      def wait_g(b):
        pltpu.make_async_copy(ysrc.at[srcv.at[0]], rows[b], semg[b]).wait()
        pltpu.make_async_copy(w_h.at[widw, 0], wv[b], semw[b]).wait()

      def wait_s(b):
        pltpu.make_async_copy(rows[b], acc.at[dstv.at[0]], sems[b]).wait()

      def scale(b):
        rb = rows[b]
        wb = wv[b]

        @plsc.parallel_loop(0, CH, unroll=8)
        def _(e):
          we = wb[e, pl.ds(0, L)]
          for k in range(nseg):
            rb[e, pl.ds(k * L, L)] = rb[e, pl.ds(k * L, L)] * we

      def phase(j, b, first=False, last=False):
        wait_g(b)
        if not first:
          wait_s((b + 1) % 2)
        if not last:
          nxt = j + 1
          if isinstance(nxt, int):
            if nxt < NCHT:
              start_g(nxt, (b + 1) % 2)
          else:
            @pl.when(nxt < NCHT)
            def _():
              start_g(nxt, (b + 1) % 2)
        scale(b)
        pltpu.async_copy(rows[b], acc.at[dstv.at[j]], sems[b], add=True)

      start_g(0, 0)
      phase(0, 0, first=True)

      def outer(t, carry):
        jbase = 1 + 2 * t
        phase(jbase, 1)
        phase(jbase + 1, 0)
        return carry

      lax.fori_loop(0, (NCHT - 2) // 2, outer, 0)
      phase(NCHT - 1, (NCHT - 1) % 2, last=True)
      wait_s((NCHT - 1) % 2)

    plsc.subcore_barrier()
    run_edges(y, rowsA, nseg1, widA, widA, accA)
    plsc.subcore_barrier()
    pltpu.sync_copy(accA.at[pl.ds(r0, RPT)],
                    outA.at[pl.ds(c * N_PAD + r0, RPT)])
    if out16:
      pltpu.sync_copy(accA.at[pl.ds(r0, RPT), pl.ds(0, 16)],
                      outA16.at[pl.ds(c * N_PAD + r0, RPT)])
    plsc.subcore_barrier()
    yB = outA16 if out16 else outA
    run_edges(yB, rowsB, nseg2, c * NW + widA, widA, accB)
    run_edges(yB, rowsB, nseg2, c * NW + widB, widB, accB)
    plsc.subcore_barrier()
    pltpu.sync_copy(accB.at[pl.ds(r0, RPT)], outB.at[c, pl.ds(r0, RPT)])

  return pl.kernel(
      body,
      out_type=tuple(outs),
      mesh=_MESH,
      scratch_types=scratch,
      compiler_params=pltpu.CompilerParams(use_tc_tiling_on_sc=False),
  )


_PROP2S = {}


def _prop2(d1, d2, out16):
  key = (d1, d2, out16)
  if key not in _PROP2S:
    _PROP2S[key] = _make_prop2(d1, d2, out16)
  return _PROP2S[key]


def _make_norm():
  scratch = [
      pltpu.VMEM((NCHT, CH), jnp.int32),
      pltpu.VMEM((NCHT, CH), jnp.float32),
      pltpu.VMEM((CH,), jnp.float32),
      pltpu.SemaphoreType.DMA,
  ]

  def body(recip_h, dst_h, ew_h, wout, dstv, wv, rbuf, sem):
    c = lax.axis_index("c")
    s = lax.axis_index("s")
    wid = c * NS + s
    pltpu.sync_copy(dst_h.at[wid], dstv)
    pltpu.sync_copy(ew_h.at[wid], wv)

    def chunk(j, carry):
      pltpu.async_copy(recip_h.at[dstv.at[j]], rbuf, sem).wait()
      for k in range(CH // L):
        wv[j, pl.ds(k * L, L)] = wv[j, pl.ds(k * L, L)] * rbuf[pl.ds(k * L, L)]
      return carry

    lax.fori_loop(0, NCHT, chunk, 0)
    pltpu.sync_copy(wv, wout.at[wid])

  return pl.kernel(
      body,
      out_type=jax.ShapeDtypeStruct((NW, NCHT, CH), jnp.float32),
      mesh=_MESH,
      scratch_types=scratch,
      compiler_params=pltpu.CompilerParams(use_tc_tiling_on_sc=False),
  )


_PROPS = {}


def _prop(d, nh):
  key = (d, nh)
  if key not in _PROPS:
    _PROPS[key] = _make_prop(d, nh)
  return _PROPS[key]


_NORM = _make_norm()


def _dot(a, b):
  return jax.lax.dot_general(a, b, (((1,), (0,)), ((), ())),
                             preferred_element_type=jnp.float32)


def _tc_recip_body(deg_h, recip_o):
  deg = deg_h[0, 0:N, 0:1] + deg_h[1, 0:N, 0:1]
  recip_o[...] = (1.0 / jnp.maximum(deg, 1e-6))[:, 0]


def _tc_recip(deg_h):
  return pl.pallas_call(
      _tc_recip_body,
      out_shape=jax.ShapeDtypeStruct((N,), jnp.float32),
  )(deg_h)


def _row(d):
  return pl.BlockSpec((RB, d), lambda i: (i, 0))


def _full(*shape):
  return pl.BlockSpec(shape, lambda i: tuple(0 for _ in shape))


def _chalf(d):
  return pl.BlockSpec((2, RB, d), lambda i: (0, i, 0))


def _tc_a_body(h, Wf, bf, Wi0, bi, xs1_o, y1_o):
  xs1 = _dot(h[...], Wf[...]) + bf[...]
  z = _dot(xs1, Wi0[...]) + bi[...]
  xs1_o[...] = xs1
  y1_o[...] = jnp.concatenate([z, h[...]], axis=-1)


def _tc_a(h, Wf, bf, Wi0, bi):
  return pl.pallas_call(
      _tc_a_body,
      grid=(NRB,),
      in_specs=[_row(H), _full(H, H), _full(1, H), _full(H, H), _full(1, H)],
      out_specs=(_row(H), _row(2 * H)),
      out_shape=(jax.ShapeDtypeStruct((N, H), jnp.float32),
                 jax.ShapeDtypeStruct((N, 2 * H), jnp.float32)),
  )(h, Wf, bf, Wi0, bi)


def _tc_b_body(C1, h, Wg, bg, Wo0, Wo1, bo, alpha, Wro0, Wro1, bro,
               rep_o, xs2_o, y2_o):
  Az = C1[0, :, 0:H] + C1[1, :, 0:H]
  Ah = C1[0, :, H:2 * H] + C1[1, :, H:2 * H]
  conv = _dot(Az, Wg[...]) + bg[...]
  o1 = _dot(conv, Wo0[...]) + _dot(h[...], Wo1[...]) + bo[...]
  out = jnp.where(o1 > 0, o1, alpha[0, 0] * o1)
  rep_o[...] = jnp.concatenate([out, h[...]], axis=-1)
  xs2 = _dot(out, Wro0[...]) + _dot(h[...], Wro1[...]) + bro[...]
  xs2_o[...] = xs2
  y2_o[...] = jnp.concatenate([xs2, Ah], axis=-1)


def _tc_b(C1, h, Wg, bg, Wo0, Wo1, bo, alpha, Wro0, Wro1, bro):
  return pl.pallas_call(
      _tc_b_body,
      grid=(NRB,),
      in_specs=[_chalf(2 * H), _row(H), _full(H, H), _full(1, H),
                _full(H, H), _full(H, H), _full(1, H), _full(1, 1),
                _full(H, IN), _full(H, IN), _full(1, IN)],
      out_specs=(_row(2 * H), _row(IN), _row(IN + H)),
      out_shape=(jax.ShapeDtypeStruct((N, 2 * H), jnp.float32),
                 jax.ShapeDtypeStruct((N, IN), jnp.float32),
                 jax.ShapeDtypeStruct((N, IN + H), jnp.float32)),
  )(C1, h, Wg, bg, Wo0, Wo1, bo, alpha, Wro0, Wro1, bro)


def _tc_c_body(C2, xs2, h, y2,
               Wr0a, Wr0b, Wr1a, Wr1b, Wr2b, br,
               Wu0a, Wu0b, Wu1a, Wu1b, Wu2b, bu,
               axs2_o, prer_o, preu_o):
  Axs2 = C2[0, :, 0:IN] + C2[1, :, 0:IN]
  A2h = C2[0, :, IN:IN + H] + C2[1, :, IN:IN + H]
  Ah = y2[:, IN:IN + H]
  xs2v = xs2[...]
  hv = h[...]
  prer = (_dot(xs2v, Wr0a[...]) + _dot(hv, Wr0b[...]) + _dot(Axs2, Wr1a[...])
          + _dot(Ah, Wr1b[...]) + _dot(A2h, Wr2b[...]) + br[...])
  preu = (_dot(xs2v, Wu0a[...]) + _dot(hv, Wu0b[...]) + _dot(Axs2, Wu1a[...])
          + _dot(Ah, Wu1b[...]) + _dot(A2h, Wu2b[...]) + bu[...])
  axs2_o[...] = Axs2
  prer_o[...] = prer
  preu_o[...] = preu


def _tc_c(C2, xs2, h, y2, wr, wu):
  wspecs = [_full(IN, H), _full(H, H), _full(IN, H), _full(H, H),
            _full(H, H), _full(1, H)]
  return pl.pallas_call(
      _tc_c_body,
      grid=(NRB,),
      in_specs=[_chalf(IN + H), _row(IN), _row(H), _row(IN + H)]
               + wspecs + wspecs,
      out_specs=(_row(IN), _row(H), _row(H)),
      out_shape=(jax.ShapeDtypeStruct((N, IN), jnp.float32),
                 jax.ShapeDtypeStruct((N, H), jnp.float32),
                 jax.ShapeDtypeStruct((N, H), jnp.float32)),
  )(C2, xs2, h, y2, *wr, *wu)


def _tc_d_body(C3, prer, preu, h, xs2, Axs2,
               Wr2a, Wu2a, Wc0a, Wc0b, Wc1a, Wc2a, bc,
               rh_o, prec_o, u_o):
  A2xs2 = C3[0] + C3[1]
  r = jax.nn.sigmoid(prer[...] + _dot(A2xs2, Wr2a[...]))
  u = jax.nn.sigmoid(preu[...] + _dot(A2xs2, Wu2a[...]))
  rh = r * h[...]
  prec = (_dot(xs2[...], Wc0a[...]) + _dot(rh, Wc0b[...])
          + _dot(Axs2[...], Wc1a[...]) + _dot(A2xs2, Wc2a[...]) + bc[...])
  rh_o[...] = rh
  prec_o[...] = prec
  u_o[...] = u


def _tc_d(C3, prer, preu, h, xs2, Axs2, Wr2a, Wu2a, Wc0a, Wc0b, Wc1a, Wc2a, bc):
  return pl.pallas_call(
      _tc_d_body,
      grid=(NRB,),
      in_specs=[_chalf(IN), _row(H), _row(H), _row(H), _row(IN), _row(IN),
                _full(IN, H), _full(IN, H), _full(IN, H), _full(H, H),
                _full(IN, H), _full(IN, H), _full(1, H)],
      out_specs=(_row(H), _row(H), _row(H)),
      out_shape=(jax.ShapeDtypeStruct((N, H), jnp.float32),
                 jax.ShapeDtypeStruct((N, H), jnp.float32),
                 jax.ShapeDtypeStruct((N, H), jnp.float32)),
  )(C3, prer, preu, h, xs2, Axs2, Wr2a, Wu2a, Wc0a, Wc0b, Wc1a, Wc2a, bc)


def _tc_f_body(C4, C5, prec, u, h, Wc1b, Wc2b, hnew_o):
  Arh = C4[0] + C4[1]
  A2rh = C5[0] + C5[1]
  c = jnp.tanh(prec[...] + _dot(Arh, Wc1b[...]) + _dot(A2rh, Wc2b[...]))
  uv = u[...]
  hnew_o[...] = uv * h[...] + (1.0 - uv) * c


def _tc_f(C4, C5, prec, u, h, Wc1b, Wc2b):
  return pl.pallas_call(
      _tc_f_body,
      grid=(NRB,),
      in_specs=[_chalf(H), _chalf(H), _row(H), _row(H), _row(H),
                _full(H, H), _full(H, H)],
      out_specs=_row(H),
      out_shape=jax.ShapeDtypeStruct((N, H), jnp.float32),
  )(C4, C5, prec, u, h, Wc1b, Wc2b)


def kernel(x, edge_index, edge_weight, Wr, br, Wu, bu, Wc, bc, Wf, bf,
           Wi, bi, Wg, bg, Wo, bo, alpha, Wro, bro):
  S = x.shape[1]

  src = edge_index[0].astype(jnp.int32)
  dst = edge_index[1].astype(jnp.int32)
  npad = E_PAD - E
  pad_idx = (jnp.arange(npad, dtype=jnp.int32) * 37) % N
  extra = ((jnp.arange(NW * 2 * CH, dtype=jnp.int32) * 37) % N
           ).reshape(NW, 2, CH)
  src_p = jnp.concatenate([
      jnp.concatenate([src, pad_idx]).reshape(NW, NCHUNK, CH), extra], axis=1)
  dst_p = jnp.concatenate([
      jnp.concatenate([dst, pad_idx]).reshape(NW, NCHUNK, CH), extra], axis=1)
  ew_p = jnp.concatenate([
      jnp.concatenate(
          [edge_weight.astype(jnp.float32), jnp.zeros((npad,), jnp.float32)]
      ).reshape(NW, NCHUNK, CH),
      jnp.zeros((NW, 2, CH), jnp.float32)], axis=1)
  src2_p = jnp.concatenate([src_p, src_p + N_PAD], axis=0)

  z16 = jnp.zeros((N_PAD, 16), jnp.float32)
  z64 = jnp.zeros((N_PAD, 64), jnp.float32)
  z80 = jnp.zeros((N_PAD, 80), jnp.float32)
  z128 = jnp.zeros((N_PAD, 128), jnp.float32)
  ones16 = jnp.ones((N, 16), jnp.float32)

  _rep = lambda t: jnp.broadcast_to(
      t.reshape(E_TAB, 1), (E_TAB, L)).reshape(NW, NCHT, CH, L)
  ew_rep = _rep(ew_p)
  deg_h = _prop(16, 1)(ones16, src_p, dst_p, ew_rep, z16)
  recip = _tc_recip(deg_h)
  wT = _NORM(recip, dst_p, ew_p)
  w_rep = _rep(wT)

  b2 = lambda b: b.reshape(1, -1)
  Wi0 = Wi[0:H]
  Wo0, Wo1 = Wo[0:H], Wo[H:2 * H]
  Wro0, Wro1 = Wro[0:H], Wro[H:2 * H]
  wr = (Wr[0:IN], Wr[32:96], Wr[96:112], Wr[128:192], Wr[224:288], b2(br))
  wu = (Wu[0:IN], Wu[32:96], Wu[96:112], Wu[128:192], Wu[224:288], b2(bu))
  Wr2a, Wu2a = Wr[192:208], Wu[192:208]
  Wc0a, Wc0b, Wc1a, Wc2a = Wc[0:IN], Wc[32:96], Wc[96:112], Wc[192:208]
  Wc1b, Wc2b = Wc[128:192], Wc[224:288]
  alpha2 = alpha.reshape(1, 1)

  h = jnp.zeros((N, H), jnp.float32)
  gens, preds, reprs, states = [], [], [], []
  for _ in range(S):
    xs1, y1 = _tc_a(h, Wf, b2(bf), Wi0, b2(bi))
    C1 = _prop(128, 1)(y1, src_p, dst_p, w_rep, z128)
    rep, xs2, y2 = _tc_b(C1, h, Wg, b2(bg), Wo0, Wo1, b2(bo), alpha2,
                         Wro0, Wro1, b2(bro))
    C2f, C3, _ = _prop2(80, 16, True)(y2, src2_p, dst_p, w_rep, z80, z16)
    C2 = C2f.reshape(2, N_PAD, 80)
    Axs2, prer, preu = _tc_c(C2, xs2, h, y2, wr, wu)
    rh, prec, u = _tc_d(C3, prer, preu, h, xs2, Axs2,
                        Wr2a, Wu2a, Wc0a, Wc0b, Wc1a, Wc2a, b2(bc))
    C4 = _prop(64, 1)(rh, src_p, dst_p, w_rep, z64)
    C5 = _prop(64, 2)(C4.reshape(2 * N_PAD, H), src2_p, dst_p, w_rep, z64)
    h = _tc_f(C4, C5, prec, u, h, Wc1b, Wc2b)
    gens.append(xs2)
    preds.append(xs1)
    reprs.append(rep)
    states.append(h)

  generations = jnp.stack(gens, 0)[None]
  predictions = jnp.stack(preds, 0)[None]
  representations = jnp.stack(reprs, 0)[None]
  states_out = jnp.stack(states, 0)[None, None]
  return generations, predictions, representations, states_out


# speedup vs baseline: 1.1962x; 1.1180x over previous
import jax
import jax.numpy as jnp
from jax import lax
from jax.experimental import pallas as pl
from jax.experimental.pallas import tpu as pltpu
from jax.experimental.pallas import tpu_sc as plsc

N = 10000
E = 160000
H = 64
IN = 16

NC, NS, L = 2, 16, 16
NW = NC * NS
CH = 128
EPW = E // NW
NCHUNK = (EPW + CH - 1) // CH
EPW_PAD = NCHUNK * CH
E_PAD = EPW_PAD * NW
NCHT = NCHUNK + 4
E_TAB = NCHT * CH * NW
N_PAD = 10240
RPT = N_PAD // NS
RB = 2000
NRB = N // RB

_MESH = plsc.VectorSubcoreMesh(core_axis_name="c", subcore_axis_name="s")


def _make_prop(d, nh):
  nseg = d // L
  nbuf = 2 if d > 80 or nh == 2 else 4
  assert NCHT % nbuf == 0

  scratch = [
      pltpu.VMEM((NCHT, CH), jnp.int32),
      pltpu.VMEM((NCHT, CH), jnp.int32),
      pltpu.VMEM_SHARED((N_PAD, d), jnp.float32),
  ]
  scratch += [pltpu.VMEM((CH, d), jnp.float32) for _ in range(nbuf)]
  scratch += [pltpu.VMEM((CH, L), jnp.float32) for _ in range(nbuf)]
  scratch += [pltpu.SemaphoreType.DMA for _ in range(3 * nbuf)]
  if nh == 2:
    scratch += [pltpu.VMEM((NCHT, CH), jnp.int32)]
    scratch += [pltpu.VMEM((CH, d), jnp.float32) for _ in range(nbuf)]
    scratch += [pltpu.SemaphoreType.DMA for _ in range(nbuf)]

  def body(y, src_h, dst_h, w_h, zeros_h, out, *rest):
    srcv, dstv, acc = rest[0], rest[1], rest[2]
    rows = rest[3:3 + nbuf]
    wv = rest[3 + nbuf:3 + 2 * nbuf]
    semg = rest[3 + 2 * nbuf:3 + 3 * nbuf]
    sems = rest[3 + 3 * nbuf:3 + 4 * nbuf]
    semw = rest[3 + 4 * nbuf:3 + 5 * nbuf]
    if nh == 2:
      src2v = rest[3 + 5 * nbuf]
      rows2 = rest[4 + 5 * nbuf:4 + 6 * nbuf]
      semg2 = rest[4 + 6 * nbuf:4 + 7 * nbuf]
    c = lax.axis_index("c")
    s = lax.axis_index("s")
    wid = c * NS + s
    pltpu.sync_copy(src_h.at[wid], srcv)
    pltpu.sync_copy(dst_h.at[wid], dstv)
    if nh == 2:
      pltpu.sync_copy(src_h.at[NW + wid], src2v)
    r0 = s * RPT
    pltpu.sync_copy(zeros_h.at[pl.ds(r0, RPT)], acc.at[pl.ds(r0, RPT)])
    plsc.subcore_barrier()

    def start_g(j, b):
      pltpu.async_copy(y.at[srcv.at[j]], rows[b], semg[b])
      pltpu.async_copy(w_h.at[wid, j], wv[b], semw[b])
      if nh == 2:
        pltpu.async_copy(y.at[src2v.at[j]], rows2[b], semg2[b])

    def wait_g(b):
      pltpu.make_async_copy(y.at[srcv.at[0]], rows[b], semg[b]).wait()
      pltpu.make_async_copy(w_h.at[wid, 0], wv[b], semw[b]).wait()
      if nh == 2:
        pltpu.make_async_copy(y.at[src2v.at[0]], rows2[b], semg2[b]).wait()

    def wait_s(b):
      pltpu.make_async_copy(rows[b], acc.at[dstv.at[0]], sems[b]).wait()

    def scale(b):
      rb = rows[b]
      r2b = rows2[b] if nh == 2 else None
      wb = wv[b]

      @plsc.parallel_loop(0, CH, unroll=8)
      def _(e):
        we = wb[e, pl.ds(0, L)]
        for k in range(nseg):
          seg = rb[e, pl.ds(k * L, L)]
          if nh == 2:
            seg = seg + r2b[e, pl.ds(k * L, L)]
          rb[e, pl.ds(k * L, L)] = seg * we

    def prefetch(j, b):
      nxt = j + nbuf - 1
      if isinstance(nxt, int):
        if nxt < NCHT:
          start_g(nxt, (b + nbuf - 1) % nbuf)
      else:
        @pl.when(nxt < NCHT)
        def _():
          start_g(nxt, (b + nbuf - 1) % nbuf)

    def phase(j, b, first=False, last=False):
      wait_g(b)
      if nbuf == 2:
        if not first:
          wait_s((b + 1) % 2)
        if not last:
          prefetch(j, b)
        scale(b)
        pltpu.async_copy(rows[b], acc.at[dstv.at[j]], sems[b], add=True)
      else:
        scale(b)
        pltpu.async_copy(rows[b], acc.at[dstv.at[j]], sems[b], add=True)
        if not first:
          wait_s((b + nbuf - 1) % nbuf)
        if not last:
          prefetch(j, b)

    for j0 in range(nbuf - 1):
      start_g(j0, j0)
    phase(0, 0, first=True)
    for j0 in range(1, nbuf - 1):
      phase(j0, j0)

    def outer(t, carry):
      jbase = (nbuf - 1) + nbuf * t
      for p in range(nbuf):
        phase(jbase + p, (nbuf - 1 + p) % nbuf)
      return carry

    lax.fori_loop(0, (NCHT - nbuf) // nbuf, outer, 0)
    phase(NCHT - 1, (NCHT - 1) % nbuf, last=True)
    wait_s((NCHT - 1) % nbuf)
    plsc.subcore_barrier()
    pltpu.sync_copy(acc.at[pl.ds(r0, RPT)], out.at[c, pl.ds(r0, RPT)])

  return pl.kernel(
      body,
      out_type=jax.ShapeDtypeStruct((2, N_PAD, d), jnp.float32),
      mesh=_MESH,
      scratch_types=scratch,
      compiler_params=pltpu.CompilerParams(use_tc_tiling_on_sc=False),
  )


def _make_prop2(d1, d2, out16):
  nseg1, nseg2 = d1 // L, d2 // L
  nbuf = 2
  same = (d1 == d2) and not out16

  scratch = [
      pltpu.VMEM((NCHT, CH), jnp.int32),
      pltpu.VMEM((NCHT, CH), jnp.int32),
      pltpu.VMEM_SHARED((N_PAD, d1), jnp.float32),
      pltpu.VMEM_SHARED((N_PAD, d2), jnp.float32),
  ]
  scratch += [pltpu.VMEM((CH, d1), jnp.float32) for _ in range(nbuf)]
  if not same:
    scratch += [pltpu.VMEM((CH, d2), jnp.float32) for _ in range(nbuf)]
  scratch += [pltpu.VMEM((CH, L), jnp.float32) for _ in range(nbuf)]
  scratch += [pltpu.SemaphoreType.DMA for _ in range(3 * nbuf)]

  outs = [jax.ShapeDtypeStruct((2 * N_PAD, d1), jnp.float32),
          jax.ShapeDtypeStruct((2, N_PAD, d2), jnp.float32)]
  if out16:
    outs.append(jax.ShapeDtypeStruct((2 * N_PAD, 16), jnp.float32))

  def body(y, src2_h, dst_h, w_h, zA, zB, *rest):
    outA, outB = rest[0], rest[1]
    rest = rest[2:]
    if out16:
      outA16 = rest[0]
      rest = rest[1:]
    srcv, dstv, accA, accB = rest[0], rest[1], rest[2], rest[3]
    rest = rest[4:]
    rowsA = rest[:nbuf]
    rest = rest[nbuf:]
    if same:
      rowsB = rowsA
    else:
      rowsB = rest[:nbuf]
      rest = rest[nbuf:]
    wv = rest[:nbuf]
    semg = rest[nbuf:2 * nbuf]
    sems = rest[2 * nbuf:3 * nbuf]
    semw = rest[3 * nbuf:4 * nbuf]
    c = lax.axis_index("c")
    s = lax.axis_index("s")
    widA = c * NS + s
    widB = (1 - c) * NS + s
    r0 = s * RPT
    pltpu.sync_copy(zA.at[pl.ds(r0, RPT)], accA.at[pl.ds(r0, RPT)])
    pltpu.sync_copy(zB.at[pl.ds(r0, RPT)], accB.at[pl.ds(r0, RPT)])

    def run_edges(ysrc, rows, nseg, src_row, widw, acc):
      pltpu.sync_copy(src2_h.at[src_row], srcv)
      pltpu.sync_copy(dst_h.at[widw], dstv)

      def start_g(j, b):
        pltpu.async_copy(ysrc.at[srcv.at[j]], rows[b], semg[b])
        pltpu.async_copy(w_h.at[widw, j], wv[b], semw[b])

      def wait_g(b):
        pltpu.make_async_copy(ysrc.at[srcv.at[0]], rows[b], semg[b]).wait()
        pltpu.make_async_copy(w_h.at[widw, 0], wv[b], semw[b]).wait()

      def wait_s(b):
        pltpu.make_async_copy(rows[b], acc.at[dstv.at[0]], sems[b]).wait()

      def scale(b):
        rb = rows[b]
        wb = wv[b]

        @plsc.parallel_loop(0, CH, unroll=8)
        def _(e):
          we = wb[e, pl.ds(0, L)]
          for k in range(nseg):
            rb[e, pl.ds(k * L, L)] = rb[e, pl.ds(k * L, L)] * we

      def phase(j, b, first=False, last=False):
        wait_g(b)
        if not first:
          wait_s((b + 1) % 2)
        if not last:
          nxt = j + 1
          if isinstance(nxt, int):
            if nxt < NCHT:
              start_g(nxt, (b + 1) % 2)
          else:
            @pl.when(nxt < NCHT)
            def _():
              start_g(nxt, (b + 1) % 2)
        scale(b)
        pltpu.async_copy(rows[b], acc.at[dstv.at[j]], sems[b], add=True)

      start_g(0, 0)
      phase(0, 0, first=True)

      def outer(t, carry):
        jbase = 1 + 2 * t
        phase(jbase, 1)
        phase(jbase + 1, 0)
        return carry

      lax.fori_loop(0, (NCHT - 2) // 2, outer, 0)
      phase(NCHT - 1, (NCHT - 1) % 2, last=True)
      wait_s((NCHT - 1) % 2)

    plsc.subcore_barrier()
    run_edges(y, rowsA, nseg1, widA, widA, accA)
    plsc.subcore_barrier()
    pltpu.sync_copy(accA.at[pl.ds(r0, RPT)],
                    outA.at[pl.ds(c * N_PAD + r0, RPT)])
    if out16:
      pltpu.sync_copy(accA.at[pl.ds(r0, RPT), pl.ds(0, 16)],
                      outA16.at[pl.ds(c * N_PAD + r0, RPT)])
    plsc.subcore_barrier()
    yB = outA16 if out16 else outA
    run_edges(yB, rowsB, nseg2, c * NW + widA, widA, accB)
    run_edges(yB, rowsB, nseg2, c * NW + widB, widB, accB)
    plsc.subcore_barrier()
    pltpu.sync_copy(accB.at[pl.ds(r0, RPT)], outB.at[c, pl.ds(r0, RPT)])

  return pl.kernel(
      body,
      out_type=tuple(outs),
      mesh=_MESH,
      scratch_types=scratch,
      compiler_params=pltpu.CompilerParams(use_tc_tiling_on_sc=False),
  )


_PROP2S = {}


def _prop2(d1, d2, out16):
  key = (d1, d2, out16)
  if key not in _PROP2S:
    _PROP2S[key] = _make_prop2(d1, d2, out16)
  return _PROP2S[key]


def _make_norm():
  scratch = [
      pltpu.VMEM((NCHT, CH), jnp.int32),
      pltpu.VMEM((NCHT, CH), jnp.float32),
      pltpu.VMEM((CH,), jnp.float32),
      pltpu.SemaphoreType.DMA,
  ]

  def body(recip_h, dst_h, ew_h, wout, dstv, wv, rbuf, sem):
    c = lax.axis_index("c")
    s = lax.axis_index("s")
    wid = c * NS + s
    pltpu.sync_copy(dst_h.at[wid], dstv)
    pltpu.sync_copy(ew_h.at[wid], wv)

    def chunk(j, carry):
      pltpu.async_copy(recip_h.at[dstv.at[j]], rbuf, sem).wait()
      for k in range(CH // L):
        wv[j, pl.ds(k * L, L)] = wv[j, pl.ds(k * L, L)] * rbuf[pl.ds(k * L, L)]
      return carry

    lax.fori_loop(0, NCHT, chunk, 0)
    pltpu.sync_copy(wv, wout.at[wid])

  return pl.kernel(
      body,
      out_type=jax.ShapeDtypeStruct((NW, NCHT, CH), jnp.float32),
      mesh=_MESH,
      scratch_types=scratch,
      compiler_params=pltpu.CompilerParams(use_tc_tiling_on_sc=False),
  )


_PROPS = {}


def _prop(d, nh):
  key = (d, nh)
  if key not in _PROPS:
    _PROPS[key] = _make_prop(d, nh)
  return _PROPS[key]


_NORM = _make_norm()


def _dot(a, b):
  return jax.lax.dot_general(a, b, (((1,), (0,)), ((), ())),
                             preferred_element_type=jnp.float32)


def _tc_recip_body(deg_h, recip_o):
  deg = deg_h[0, 0:N, 0:1] + deg_h[1, 0:N, 0:1]
  recip_o[...] = (1.0 / jnp.maximum(deg, 1e-6))[:, 0]


def _tc_recip(deg_h):
  return pl.pallas_call(
      _tc_recip_body,
      out_shape=jax.ShapeDtypeStruct((N,), jnp.float32),
  )(deg_h)


def _row(d):
  return pl.BlockSpec((RB, d), lambda i: (i, 0))


def _full(*shape):
  return pl.BlockSpec(shape, lambda i: tuple(0 for _ in shape))


def _chalf(d):
  return pl.BlockSpec((2, RB, d), lambda i: (0, i, 0))


def _tc_a_body(h, Wf, bf, Wi0, bi, xs1_o, y1_o):
  xs1 = _dot(h[...], Wf[...]) + bf[...]
  z = _dot(xs1, Wi0[...]) + bi[...]
  xs1_o[...] = xs1
  y1_o[...] = jnp.concatenate([z, h[...]], axis=-1)


def _tc_a(h, Wf, bf, Wi0, bi):
  return pl.pallas_call(
      _tc_a_body,
      grid=(NRB,),
      in_specs=[_row(H), _full(H, H), _full(1, H), _full(H, H), _full(1, H)],
      out_specs=(_row(H), _row(2 * H)),
      out_shape=(jax.ShapeDtypeStruct((N, H), jnp.float32),
                 jax.ShapeDtypeStruct((N, 2 * H), jnp.float32)),
  )(h, Wf, bf, Wi0, bi)


def _tc_b_body(C1, h, Wg, bg, Wo0, Wo1, bo, alpha, Wro0, Wro1, bro,
               rep_o, xs2_o, y2_o):
  Az = C1[0, :, 0:H] + C1[1, :, 0:H]
  Ah = C1[0, :, H:2 * H] + C1[1, :, H:2 * H]
  conv = _dot(Az, Wg[...]) + bg[...]
  o1 = _dot(conv, Wo0[...]) + _dot(h[...], Wo1[...]) + bo[...]
  out = jnp.where(o1 > 0, o1, alpha[0, 0] * o1)
  rep_o[...] = jnp.concatenate([out, h[...]], axis=-1)
  xs2 = _dot(out, Wro0[...]) + _dot(h[...], Wro1[...]) + bro[...]
  xs2_o[...] = xs2
  y2_o[...] = jnp.concatenate([xs2, Ah], axis=-1)


def _tc_b(C1, h, Wg, bg, Wo0, Wo1, bo, alpha, Wro0, Wro1, bro):
  return pl.pallas_call(
      _tc_b_body,
      grid=(NRB,),
      in_specs=[_chalf(2 * H), _row(H), _full(H, H), _full(1, H),
                _full(H, H), _full(H, H), _full(1, H), _full(1, 1),
                _full(H, IN), _full(H, IN), _full(1, IN)],
      out_specs=(_row(2 * H), _row(IN), _row(IN + H)),
      out_shape=(jax.ShapeDtypeStruct((N, 2 * H), jnp.float32),
                 jax.ShapeDtypeStruct((N, IN), jnp.float32),
                 jax.ShapeDtypeStruct((N, IN + H), jnp.float32)),
  )(C1, h, Wg, bg, Wo0, Wo1, bo, alpha, Wro0, Wro1, bro)


def _tc_c_body(C2, xs2, h, y2,
               Wr0a, Wr0b, Wr1a, Wr1b, Wr2b, br,
               Wu0a, Wu0b, Wu1a, Wu1b, Wu2b, bu,
               axs2_o, prer_o, preu_o):
  Axs2 = C2[0, :, 0:IN] + C2[1, :, 0:IN]
  A2h = C2[0, :, IN:IN + H] + C2[1, :, IN:IN + H]
  Ah = y2[:, IN:IN + H]
  xs2v = xs2[...]
  hv = h[...]
  prer = (_dot(xs2v, Wr0a[...]) + _dot(hv, Wr0b[...]) + _dot(Axs2, Wr1a[...])
          + _dot(Ah, Wr1b[...]) + _dot(A2h, Wr2b[...]) + br[...])
  preu = (_dot(xs2v, Wu0a[...]) + _dot(hv, Wu0b[...]) + _dot(Axs2, Wu1a[...])
          + _dot(Ah, Wu1b[...]) + _dot(A2h, Wu2b[...]) + bu[...])
  axs2_o[...] = Axs2
  prer_o[...] = prer
  preu_o[...] = preu


def _tc_c(C2, xs2, h, y2, wr, wu):
  wspecs = [_full(IN, H), _full(H, H), _full(IN, H), _full(H, H),
            _full(H, H), _full(1, H)]
  return pl.pallas_call(
      _tc_c_body,
      grid=(NRB,),
      in_specs=[_chalf(IN + H), _row(IN), _row(H), _row(IN + H)]
               + wspecs + wspecs,
      out_specs=(_row(IN), _row(H), _row(H)),
      out_shape=(jax.ShapeDtypeStruct((N, IN), jnp.float32),
                 jax.ShapeDtypeStruct((N, H), jnp.float32),
                 jax.ShapeDtypeStruct((N, H), jnp.float32)),
  )(C2, xs2, h, y2, *wr, *wu)


def _tc_d_body(C3, prer, preu, h, xs2, Axs2,
               Wr2a, Wu2a, Wc0a, Wc0b, Wc1a, Wc2a, bc,
               rh_o, prec_o, u_o):
  A2xs2 = C3[0] + C3[1]
  r = jax.nn.sigmoid(prer[...] + _dot(A2xs2, Wr2a[...]))
  u = jax.nn.sigmoid(preu[...] + _dot(A2xs2, Wu2a[...]))
  rh = r * h[...]
  prec = (_dot(xs2[...], Wc0a[...]) + _dot(rh, Wc0b[...])
          + _dot(Axs2[...], Wc1a[...]) + _dot(A2xs2, Wc2a[...]) + bc[...])
  rh_o[...] = rh
  prec_o[...] = prec
  u_o[...] = u


def _tc_d(C3, prer, preu, h, xs2, Axs2, Wr2a, Wu2a, Wc0a, Wc0b, Wc1a, Wc2a, bc):
  return pl.pallas_call(
      _tc_d_body,
      grid=(NRB,),
      in_specs=[_chalf(IN), _row(H), _row(H), _row(H), _row(IN), _row(IN),
                _full(IN, H), _full(IN, H), _full(IN, H), _full(H, H),
                _full(IN, H), _full(IN, H), _full(1, H)],
      out_specs=(_row(H), _row(H), _row(H)),
      out_shape=(jax.ShapeDtypeStruct((N, H), jnp.float32),
                 jax.ShapeDtypeStruct((N, H), jnp.float32),
                 jax.ShapeDtypeStruct((N, H), jnp.float32)),
  )(C3, prer, preu, h, xs2, Axs2, Wr2a, Wu2a, Wc0a, Wc0b, Wc1a, Wc2a, bc)


def _tc_f_body(C4, C5, prec, u, h, Wc1b, Wc2b, hnew_o):
  Arh = C4[0] + C4[1]
  A2rh = C5[0] + C5[1]
  c = jnp.tanh(prec[...] + _dot(Arh, Wc1b[...]) + _dot(A2rh, Wc2b[...]))
  uv = u[...]
  hnew_o[...] = uv * h[...] + (1.0 - uv) * c


def _tc_f(C4, C5, prec, u, h, Wc1b, Wc2b):
  return pl.pallas_call(
      _tc_f_body,
      grid=(NRB,),
      in_specs=[_chalf(H), _chalf(H), _row(H), _row(H), _row(H),
                _full(H, H), _full(H, H)],
      out_specs=_row(H),
      out_shape=jax.ShapeDtypeStruct((N, H), jnp.float32),
  )(C4, C5, prec, u, h, Wc1b, Wc2b)


def kernel(x, edge_index, edge_weight, Wr, br, Wu, bu, Wc, bc, Wf, bf,
           Wi, bi, Wg, bg, Wo, bo, alpha, Wro, bro):
  S = x.shape[1]

  src = edge_index[0].astype(jnp.int32)
  dst = edge_index[1].astype(jnp.int32)
  npad = E_PAD - E
  pad_idx = (jnp.arange(npad, dtype=jnp.int32) * 37) % N
  extra = ((jnp.arange(NW * 4 * CH, dtype=jnp.int32) * 37) % N
           ).reshape(NW, 4, CH)
  src_p = jnp.concatenate([
      jnp.concatenate([src, pad_idx]).reshape(NW, NCHUNK, CH), extra], axis=1)
  dst_p = jnp.concatenate([
      jnp.concatenate([dst, pad_idx]).reshape(NW, NCHUNK, CH), extra], axis=1)
  ew_p = jnp.concatenate([
      jnp.concatenate(
          [edge_weight.astype(jnp.float32), jnp.zeros((npad,), jnp.float32)]
      ).reshape(NW, NCHUNK, CH),
      jnp.zeros((NW, 4, CH), jnp.float32)], axis=1)
  src2_p = jnp.concatenate([src_p, src_p + N_PAD], axis=0)

  z16 = jnp.zeros((N_PAD, 16), jnp.float32)
  z64 = jnp.zeros((N_PAD, 64), jnp.float32)
  z80 = jnp.zeros((N_PAD, 80), jnp.float32)
  z128 = jnp.zeros((N_PAD, 128), jnp.float32)
  ones16 = jnp.ones((N, 16), jnp.float32)

  _rep = lambda t: jnp.broadcast_to(
      t.reshape(E_TAB, 1), (E_TAB, L)).reshape(NW, NCHT, CH, L)
  ew_rep = _rep(ew_p)
  deg_h = _prop(16, 1)(ones16, src_p, dst_p, ew_rep, z16)
  recip = _tc_recip(deg_h)
  wT = _NORM(recip, dst_p, ew_p)
  w_rep = _rep(wT)

  b2 = lambda b: b.reshape(1, -1)
  Wi0 = Wi[0:H]
  Wo0, Wo1 = Wo[0:H], Wo[H:2 * H]
  Wro0, Wro1 = Wro[0:H], Wro[H:2 * H]
  wr = (Wr[0:IN], Wr[32:96], Wr[96:112], Wr[128:192], Wr[224:288], b2(br))
  wu = (Wu[0:IN], Wu[32:96], Wu[96:112], Wu[128:192], Wu[224:288], b2(bu))
  Wr2a, Wu2a = Wr[192:208], Wu[192:208]
  Wc0a, Wc0b, Wc1a, Wc2a = Wc[0:IN], Wc[32:96], Wc[96:112], Wc[192:208]
  Wc1b, Wc2b = Wc[128:192], Wc[224:288]
  alpha2 = alpha.reshape(1, 1)

  h = jnp.zeros((N, H), jnp.float32)
  gens, preds, reprs, states = [], [], [], []
  for _ in range(S):
    xs1, y1 = _tc_a(h, Wf, b2(bf), Wi0, b2(bi))
    C1 = _prop(128, 1)(y1, src_p, dst_p, w_rep, z128)
    rep, xs2, y2 = _tc_b(C1, h, Wg, b2(bg), Wo0, Wo1, b2(bo), alpha2,
                         Wro0, Wro1, b2(bro))
    C2 = _prop(80, 1)(y2, src_p, dst_p, w_rep, z80)
    Axs2, prer, preu = _tc_c(C2, xs2, h, y2, wr, wu)
    C3 = _prop(16, 1)(Axs2, src_p, dst_p, w_rep, z16)
    rh, prec, u = _tc_d(C3, prer, preu, h, xs2, Axs2,
                        Wr2a, Wu2a, Wc0a, Wc0b, Wc1a, Wc2a, b2(bc))
    C4 = _prop(64, 1)(rh, src_p, dst_p, w_rep, z64)
    C5 = _prop(64, 2)(C4.reshape(2 * N_PAD, H), src2_p, dst_p, w_rep, z64)
    h = _tc_f(C4, C5, prec, u, h, Wc1b, Wc2b)
    gens.append(xs2)
    preds.append(xs1)
    reprs.append(rep)
    states.append(h)

  generations = jnp.stack(gens, 0)[None]
  predictions = jnp.stack(preds, 0)[None]
  representations = jnp.stack(reprs, 0)[None]
  states_out = jnp.stack(states, 0)[None, None]
  return generations, predictions, representations, states_out


# speedup vs baseline: 1.2002x; 1.0033x over previous
import jax
import jax.numpy as jnp
from jax import lax
from jax.experimental import pallas as pl
from jax.experimental.pallas import tpu as pltpu
from jax.experimental.pallas import tpu_sc as plsc

N = 10000
E = 160000
H = 64
IN = 16

NC, NS, L = 2, 16, 16
NW = NC * NS
CH = 128
EPW = E // NW
NCHUNK = (EPW + CH - 1) // CH
EPW_PAD = NCHUNK * CH
E_PAD = EPW_PAD * NW
NCHT = NCHUNK + 4
E_TAB = NCHT * CH * NW
N_PAD = 10240
RPT = N_PAD // NS
RB = 2000
NRB = N // RB

_MESH = plsc.VectorSubcoreMesh(core_axis_name="c", subcore_axis_name="s")


def _make_prop(d, nh):
  nseg = d // L
  nbuf = 2 if d > 80 or nh == 2 else 4
  assert NCHT % nbuf == 0

  scratch = [
      pltpu.VMEM((NCHT, CH), jnp.int32),
      pltpu.VMEM((NCHT, CH), jnp.int32),
      pltpu.VMEM_SHARED((N_PAD, d), jnp.float32),
  ]
  scratch += [pltpu.VMEM((CH, d), jnp.float32) for _ in range(nbuf)]
  scratch += [pltpu.VMEM((CH, L), jnp.float32) for _ in range(nbuf)]
  scratch += [pltpu.SemaphoreType.DMA for _ in range(3 * nbuf)]
  if nh == 2:
    scratch += [pltpu.VMEM((NCHT, CH), jnp.int32)]
    scratch += [pltpu.VMEM((CH, d), jnp.float32) for _ in range(nbuf)]
    scratch += [pltpu.SemaphoreType.DMA for _ in range(nbuf)]

  def body(y, src_h, dst_h, w_h, zeros_h, out, *rest):
    srcv, dstv, acc = rest[0], rest[1], rest[2]
    rows = rest[3:3 + nbuf]
    wv = rest[3 + nbuf:3 + 2 * nbuf]
    semg = rest[3 + 2 * nbuf:3 + 3 * nbuf]
    sems = rest[3 + 3 * nbuf:3 + 4 * nbuf]
    semw = rest[3 + 4 * nbuf:3 + 5 * nbuf]
    if nh == 2:
      src2v = rest[3 + 5 * nbuf]
      rows2 = rest[4 + 5 * nbuf:4 + 6 * nbuf]
      semg2 = rest[4 + 6 * nbuf:4 + 7 * nbuf]
    c = lax.axis_index("c")
    s = lax.axis_index("s")
    wid = c * NS + s
    pltpu.sync_copy(src_h.at[wid], srcv)
    pltpu.sync_copy(dst_h.at[wid], dstv)
    if nh == 2:
      pltpu.sync_copy(src_h.at[NW + wid], src2v)
    r0 = s * RPT
    pltpu.sync_copy(zeros_h.at[pl.ds(r0, RPT)], acc.at[pl.ds(r0, RPT)])
    plsc.subcore_barrier()

    def start_g(j, b):
      pltpu.async_copy(y.at[srcv.at[j]], rows[b], semg[b])
      pltpu.async_copy(w_h.at[wid, j], wv[b], semw[b])
      if nh == 2:
        pltpu.async_copy(y.at[src2v.at[j]], rows2[b], semg2[b])

    def wait_g(b):
      pltpu.make_async_copy(y.at[srcv.at[0]], rows[b], semg[b]).wait()
      pltpu.make_async_copy(w_h.at[wid, 0], wv[b], semw[b]).wait()
      if nh == 2:
        pltpu.make_async_copy(y.at[src2v.at[0]], rows2[b], semg2[b]).wait()

    def wait_s(b):
      pltpu.make_async_copy(rows[b], acc.at[dstv.at[0]], sems[b]).wait()

    def scale(b):
      rb = rows[b]
      r2b = rows2[b] if nh == 2 else None
      wb = wv[b]

      @plsc.parallel_loop(0, CH, unroll=8)
      def _(e):
        we = wb[e, pl.ds(0, L)]
        for k in range(nseg):
          seg = rb[e, pl.ds(k * L, L)]
          if nh == 2:
            seg = seg + r2b[e, pl.ds(k * L, L)]
          rb[e, pl.ds(k * L, L)] = seg * we

    def prefetch(j, b):
      nxt = j + nbuf - 1
      if isinstance(nxt, int):
        if nxt < NCHT:
          start_g(nxt, (b + nbuf - 1) % nbuf)
      else:
        @pl.when(nxt < NCHT)
        def _():
          start_g(nxt, (b + nbuf - 1) % nbuf)

    def phase(j, b, first=False, last=False):
      wait_g(b)
      if nbuf == 2:
        if not first:
          wait_s((b + 1) % 2)
        if not last:
          prefetch(j, b)
        scale(b)
        pltpu.async_copy(rows[b], acc.at[dstv.at[j]], sems[b], add=True)
      else:
        scale(b)
        pltpu.async_copy(rows[b], acc.at[dstv.at[j]], sems[b], add=True)
        if not first:
          wait_s((b + nbuf - 1) % nbuf)
        if not last:
          prefetch(j, b)

    for j0 in range(nbuf - 1):
      start_g(j0, j0)
    phase(0, 0, first=True)
    for j0 in range(1, nbuf - 1):
      phase(j0, j0)

    def outer(t, carry):
      jbase = (nbuf - 1) + nbuf * t
      for p in range(nbuf):
        phase(jbase + p, (nbuf - 1 + p) % nbuf)
      return carry

    lax.fori_loop(0, (NCHT - nbuf) // nbuf, outer, 0)
    phase(NCHT - 1, (NCHT - 1) % nbuf, last=True)
    wait_s((NCHT - 1) % nbuf)
    plsc.subcore_barrier()
    pltpu.sync_copy(acc.at[pl.ds(r0, RPT)], out.at[c, pl.ds(r0, RPT)])

  return pl.kernel(
      body,
      out_type=jax.ShapeDtypeStruct((2, N_PAD, d), jnp.float32),
      mesh=_MESH,
      scratch_types=scratch,
      compiler_params=pltpu.CompilerParams(use_tc_tiling_on_sc=False),
  )


def _make_prop2(d1, d2, out16):
  nseg1, nseg2 = d1 // L, d2 // L
  nbuf = 2
  same = (d1 == d2) and not out16

  scratch = [
      pltpu.VMEM((NCHT, CH), jnp.int32),
      pltpu.VMEM((NCHT, CH), jnp.int32),
      pltpu.VMEM_SHARED((N_PAD, d1), jnp.float32),
      pltpu.VMEM_SHARED((N_PAD, d2), jnp.float32),
  ]
  scratch += [pltpu.VMEM((CH, d1), jnp.float32) for _ in range(nbuf)]
  if not same:
    scratch += [pltpu.VMEM((CH, d2), jnp.float32) for _ in range(nbuf)]
  scratch += [pltpu.VMEM((CH, L), jnp.float32) for _ in range(nbuf)]
  scratch += [pltpu.SemaphoreType.DMA for _ in range(3 * nbuf)]

  outs = [jax.ShapeDtypeStruct((2 * N_PAD, d1), jnp.float32),
          jax.ShapeDtypeStruct((2, N_PAD, d2), jnp.float32)]
  if out16:
    outs.append(jax.ShapeDtypeStruct((2 * N_PAD, 16), jnp.float32))

  def body(y, src2_h, dst_h, w_h, zA, zB, *rest):
    outA, outB = rest[0], rest[1]
    rest = rest[2:]
    if out16:
      outA16 = rest[0]
      rest = rest[1:]
    srcv, dstv, accA, accB = rest[0], rest[1], rest[2], rest[3]
    rest = rest[4:]
    rowsA = rest[:nbuf]
    rest = rest[nbuf:]
    if same:
      rowsB = rowsA
    else:
      rowsB = rest[:nbuf]
      rest = rest[nbuf:]
    wv = rest[:nbuf]
    semg = rest[nbuf:2 * nbuf]
    sems = rest[2 * nbuf:3 * nbuf]
    semw = rest[3 * nbuf:4 * nbuf]
    c = lax.axis_index("c")
    s = lax.axis_index("s")
    widA = c * NS + s
    widB = (1 - c) * NS + s
    r0 = s * RPT
    pltpu.sync_copy(zA.at[pl.ds(r0, RPT)], accA.at[pl.ds(r0, RPT)])
    pltpu.sync_copy(zB.at[pl.ds(r0, RPT)], accB.at[pl.ds(r0, RPT)])

    def run_edges(ysrc, rows, nseg, src_row, widw, acc):
      pltpu.sync_copy(src2_h.at[src_row], srcv)
      pltpu.sync_copy(dst_h.at[widw], dstv)

      def start_g(j, b):
        pltpu.async_copy(ysrc.at[srcv.at[j]], rows[b], semg[b])
        pltpu.async_copy(w_h.at[widw, j], wv[b], semw[b])

      def wait_g(b):
        pltpu.make_async_copy(ysrc.at[srcv.at[0]], rows[b], semg[b]).wait()
        pltpu.make_async_copy(w_h.at[widw, 0], wv[b], semw[b]).wait()

      def wait_s(b):
        pltpu.make_async_copy(rows[b], acc.at[dstv.at[0]], sems[b]).wait()

      def scale(b):
        rb = rows[b]
        wb = wv[b]

        @plsc.parallel_loop(0, CH, unroll=8)
        def _(e):
          we = wb[e, pl.ds(0, L)]
          for k in range(nseg):
            rb[e, pl.ds(k * L, L)] = rb[e, pl.ds(k * L, L)] * we

      def phase(j, b, first=False, last=False):
        wait_g(b)
        if not first:
          wait_s((b + 1) % 2)
        if not last:
          nxt = j + 1
          if isinstance(nxt, int):
            if nxt < NCHT:
              start_g(nxt, (b + 1) % 2)
          else:
            @pl.when(nxt < NCHT)
            def _():
              start_g(nxt, (b + 1) % 2)
        scale(b)
        pltpu.async_copy(rows[b], acc.at[dstv.at[j]], sems[b], add=True)

      start_g(0, 0)
      phase(0, 0, first=True)

      def outer(t, carry):
        jbase = 1 + 2 * t
        phase(jbase, 1)
        phase(jbase + 1, 0)
        return carry

      lax.fori_loop(0, (NCHT - 2) // 2, outer, 0)
      phase(NCHT - 1, (NCHT - 1) % 2, last=True)
      wait_s((NCHT - 1) % 2)

    plsc.subcore_barrier()
    run_edges(y, rowsA, nseg1, widA, widA, accA)
    plsc.subcore_barrier()
    pltpu.sync_copy(accA.at[pl.ds(r0, RPT)],
                    outA.at[pl.ds(c * N_PAD + r0, RPT)])
    if out16:
      pltpu.sync_copy(accA.at[pl.ds(r0, RPT), pl.ds(0, 16)],
                      outA16.at[pl.ds(c * N_PAD + r0, RPT)])
    plsc.subcore_barrier()
    yB = outA16 if out16 else outA
    run_edges(yB, rowsB, nseg2, c * NW + widA, widA, accB)
    run_edges(yB, rowsB, nseg2, c * NW + widB, widB, accB)
    plsc.subcore_barrier()
    pltpu.sync_copy(accB.at[pl.ds(r0, RPT)], outB.at[c, pl.ds(r0, RPT)])

  return pl.kernel(
      body,
      out_type=tuple(outs),
      mesh=_MESH,
      scratch_types=scratch,
      compiler_params=pltpu.CompilerParams(use_tc_tiling_on_sc=False),
  )


_PROP2S = {}


def _prop2(d1, d2, out16):
  key = (d1, d2, out16)
  if key not in _PROP2S:
    _PROP2S[key] = _make_prop2(d1, d2, out16)
  return _PROP2S[key]


def _make_norm():
  scratch = [
      pltpu.VMEM((NCHT, CH), jnp.int32),
      pltpu.VMEM((NCHT, CH), jnp.float32),
      pltpu.VMEM((CH,), jnp.float32),
      pltpu.SemaphoreType.DMA,
  ]

  def body(recip_h, dst_h, ew_h, wout, dstv, wv, rbuf, sem):
    c = lax.axis_index("c")
    s = lax.axis_index("s")
    wid = c * NS + s
    pltpu.sync_copy(dst_h.at[wid], dstv)
    pltpu.sync_copy(ew_h.at[wid], wv)

    def chunk(j, carry):
      pltpu.async_copy(recip_h.at[dstv.at[j]], rbuf, sem).wait()
      for k in range(CH // L):
        wv[j, pl.ds(k * L, L)] = wv[j, pl.ds(k * L, L)] * rbuf[pl.ds(k * L, L)]
      return carry

    lax.fori_loop(0, NCHT, chunk, 0)
    pltpu.sync_copy(wv, wout.at[wid])

  return pl.kernel(
      body,
      out_type=jax.ShapeDtypeStruct((NW, NCHT, CH), jnp.float32),
      mesh=_MESH,
      scratch_types=scratch,
      compiler_params=pltpu.CompilerParams(use_tc_tiling_on_sc=False),
  )


_PROPS = {}


def _prop(d, nh):
  key = (d, nh)
  if key not in _PROPS:
    _PROPS[key] = _make_prop(d, nh)
  return _PROPS[key]


_NORM = _make_norm()


def _dot(a, b):
  return jax.lax.dot_general(a, b, (((1,), (0,)), ((), ())),
                             preferred_element_type=jnp.float32)


def _tc_recip_body(deg_h, recip_o):
  deg = deg_h[0, 0:N, 0:1] + deg_h[1, 0:N, 0:1]
  recip_o[...] = (1.0 / jnp.maximum(deg, 1e-6))[:, 0]


def _tc_recip(deg_h):
  return pl.pallas_call(
      _tc_recip_body,
      out_shape=jax.ShapeDtypeStruct((N,), jnp.float32),
  )(deg_h)


def _row(d):
  return pl.BlockSpec((RB, d), lambda i: (i, 0))


def _full(*shape):
  return pl.BlockSpec(shape, lambda i: tuple(0 for _ in shape))


def _chalf(d):
  return pl.BlockSpec((2, RB, d), lambda i: (0, i, 0))


def _tc_a_body(h, Wf, bf, Wi0, bi, xs1_o, y1_o):
  xs1 = _dot(h[...], Wf[...]) + bf[...]
  z = _dot(xs1, Wi0[...]) + bi[...]
  xs1_o[...] = xs1
  y1_o[...] = jnp.concatenate([z, h[...]], axis=-1)


def _tc_a(h, Wf, bf, Wi0, bi):
  return pl.pallas_call(
      _tc_a_body,
      grid=(NRB,),
      in_specs=[_row(H), _full(H, H), _full(1, H), _full(H, H), _full(1, H)],
      out_specs=(_row(H), _row(2 * H)),
      out_shape=(jax.ShapeDtypeStruct((N, H), jnp.float32),
                 jax.ShapeDtypeStruct((N, 2 * H), jnp.float32)),
  )(h, Wf, bf, Wi0, bi)


def _tc_b_body(C1, h, Wg, bg, Wo0, Wo1, bo, alpha, Wro0, Wro1, bro,
               rep_o, xs2_o, y2_o):
  Az = C1[0, :, 0:H] + C1[1, :, 0:H]
  Ah = C1[0, :, H:2 * H] + C1[1, :, H:2 * H]
  conv = _dot(Az, Wg[...]) + bg[...]
  o1 = _dot(conv, Wo0[...]) + _dot(h[...], Wo1[...]) + bo[...]
  out = jnp.where(o1 > 0, o1, alpha[0, 0] * o1)
  rep_o[...] = jnp.concatenate([out, h[...]], axis=-1)
  xs2 = _dot(out, Wro0[...]) + _dot(h[...], Wro1[...]) + bro[...]
  xs2_o[...] = xs2
  y2_o[...] = jnp.concatenate([xs2, Ah], axis=-1)


def _tc_b(C1, h, Wg, bg, Wo0, Wo1, bo, alpha, Wro0, Wro1, bro):
  return pl.pallas_call(
      _tc_b_body,
      grid=(NRB,),
      in_specs=[_chalf(2 * H), _row(H), _full(H, H), _full(1, H),
                _full(H, H), _full(H, H), _full(1, H), _full(1, 1),
                _full(H, IN), _full(H, IN), _full(1, IN)],
      out_specs=(_row(2 * H), _row(IN), _row(IN + H)),
      out_shape=(jax.ShapeDtypeStruct((N, 2 * H), jnp.float32),
                 jax.ShapeDtypeStruct((N, IN), jnp.float32),
                 jax.ShapeDtypeStruct((N, IN + H), jnp.float32)),
  )(C1, h, Wg, bg, Wo0, Wo1, bo, alpha, Wro0, Wro1, bro)


def _tc_c_body(C2, xs2, h, y2,
               Wr0a, Wr0b, Wr1a, Wr1b, Wr2b, br,
               Wu0a, Wu0b, Wu1a, Wu1b, Wu2b, bu,
               axs2_o, prer_o, preu_o):
  Axs2 = C2[0, :, 0:IN] + C2[1, :, 0:IN]
  A2h = C2[0, :, IN:IN + H] + C2[1, :, IN:IN + H]
  Ah = y2[:, IN:IN + H]
  xs2v = xs2[...]
  hv = h[...]
  prer = (_dot(xs2v, Wr0a[...]) + _dot(hv, Wr0b[...]) + _dot(Axs2, Wr1a[...])
          + _dot(Ah, Wr1b[...]) + _dot(A2h, Wr2b[...]) + br[...])
  preu = (_dot(xs2v, Wu0a[...]) + _dot(hv, Wu0b[...]) + _dot(Axs2, Wu1a[...])
          + _dot(Ah, Wu1b[...]) + _dot(A2h, Wu2b[...]) + bu[...])
  axs2_o[...] = Axs2
  prer_o[...] = prer
  preu_o[...] = preu


def _tc_c(C2, xs2, h, y2, wr, wu):
  wspecs = [_full(IN, H), _full(H, H), _full(IN, H), _full(H, H),
            _full(H, H), _full(1, H)]
  return pl.pallas_call(
      _tc_c_body,
      grid=(NRB,),
      in_specs=[_chalf(IN + H), _row(IN), _row(H), _row(IN + H)]
               + wspecs + wspecs,
      out_specs=(_row(IN), _row(H), _row(H)),
      out_shape=(jax.ShapeDtypeStruct((N, IN), jnp.float32),
                 jax.ShapeDtypeStruct((N, H), jnp.float32),
                 jax.ShapeDtypeStruct((N, H), jnp.float32)),
  )(C2, xs2, h, y2, *wr, *wu)


def _tc_d_body(C3, prer, preu, h, xs2, Axs2,
               Wr2a, Wu2a, Wc0a, Wc0b, Wc1a, Wc2a, bc,
               rh_o, prec_o, u_o):
  A2xs2 = C3[0] + C3[1]
  r = jax.nn.sigmoid(prer[...] + _dot(A2xs2, Wr2a[...]))
  u = jax.nn.sigmoid(preu[...] + _dot(A2xs2, Wu2a[...]))
  rh = r * h[...]
  prec = (_dot(xs2[...], Wc0a[...]) + _dot(rh, Wc0b[...])
          + _dot(Axs2[...], Wc1a[...]) + _dot(A2xs2, Wc2a[...]) + bc[...])
  rh_o[...] = rh
  prec_o[...] = prec
  u_o[...] = u


def _tc_d(C3, prer, preu, h, xs2, Axs2, Wr2a, Wu2a, Wc0a, Wc0b, Wc1a, Wc2a, bc):
  return pl.pallas_call(
      _tc_d_body,
      grid=(NRB,),
      in_specs=[_chalf(IN), _row(H), _row(H), _row(H), _row(IN), _row(IN),
                _full(IN, H), _full(IN, H), _full(IN, H), _full(H, H),
                _full(IN, H), _full(IN, H), _full(1, H)],
      out_specs=(_row(H), _row(H), _row(H)),
      out_shape=(jax.ShapeDtypeStruct((N, H), jnp.float32),
                 jax.ShapeDtypeStruct((N, H), jnp.float32),
                 jax.ShapeDtypeStruct((N, H), jnp.float32)),
  )(C3, prer, preu, h, xs2, Axs2, Wr2a, Wu2a, Wc0a, Wc0b, Wc1a, Wc2a, bc)


def _tc_f_body(C4, C5, prec, u, h, Wc1b, Wc2b, hnew_o):
  Arh = C4[0] + C4[1]
  A2rh = C5[0] + C5[1]
  c = jnp.tanh(prec[...] + _dot(Arh, Wc1b[...]) + _dot(A2rh, Wc2b[...]))
  uv = u[...]
  hnew_o[...] = uv * h[...] + (1.0 - uv) * c


def _tc_f(C4, C5, prec, u, h, Wc1b, Wc2b):
  return pl.pallas_call(
      _tc_f_body,
      grid=(NRB,),
      in_specs=[_chalf(H), _chalf(H), _row(H), _row(H), _row(H),
                _full(H, H), _full(H, H)],
      out_specs=_row(H),
      out_shape=jax.ShapeDtypeStruct((N, H), jnp.float32),
  )(C4, C5, prec, u, h, Wc1b, Wc2b)


def _tc_fa_body(C4, C5, prec, u, h, Wc1b, Wc2b, Wf, bf, Wi0, bi,
                hnew_o, xs1_o, y1_o):
  Arh = C4[0] + C4[1]
  A2rh = C5[0] + C5[1]
  c = jnp.tanh(prec[...] + _dot(Arh, Wc1b[...]) + _dot(A2rh, Wc2b[...]))
  uv = u[...]
  hnew = uv * h[...] + (1.0 - uv) * c
  hnew_o[...] = hnew
  xs1 = _dot(hnew, Wf[...]) + bf[...]
  z = _dot(xs1, Wi0[...]) + bi[...]
  xs1_o[...] = xs1
  y1_o[...] = jnp.concatenate([z, hnew], axis=-1)


def _tc_fa(C4, C5, prec, u, h, Wc1b, Wc2b, Wf, bf, Wi0, bi):
  return pl.pallas_call(
      _tc_fa_body,
      grid=(NRB,),
      in_specs=[_chalf(H), _chalf(H), _row(H), _row(H), _row(H),
                _full(H, H), _full(H, H), _full(H, H), _full(1, H),
                _full(H, H), _full(1, H)],
      out_specs=(_row(H), _row(H), _row(2 * H)),
      out_shape=(jax.ShapeDtypeStruct((N, H), jnp.float32),
                 jax.ShapeDtypeStruct((N, H), jnp.float32),
                 jax.ShapeDtypeStruct((N, 2 * H), jnp.float32)),
  )(C4, C5, prec, u, h, Wc1b, Wc2b, Wf, bf, Wi0, bi)


def kernel(x, edge_index, edge_weight, Wr, br, Wu, bu, Wc, bc, Wf, bf,
           Wi, bi, Wg, bg, Wo, bo, alpha, Wro, bro):
  S = x.shape[1]

  src = edge_index[0].astype(jnp.int32)
  dst = edge_index[1].astype(jnp.int32)
  npad = E_PAD - E
  pad_idx = (jnp.arange(npad, dtype=jnp.int32) * 37) % N
  extra = ((jnp.arange(NW * 4 * CH, dtype=jnp.int32) * 37) % N
           ).reshape(NW, 4, CH)
  src_p = jnp.concatenate([
      jnp.concatenate([src, pad_idx]).reshape(NW, NCHUNK, CH), extra], axis=1)
  dst_p = jnp.concatenate([
      jnp.concatenate([dst, pad_idx]).reshape(NW, NCHUNK, CH), extra], axis=1)
  ew_p = jnp.concatenate([
      jnp.concatenate(
          [edge_weight.astype(jnp.float32), jnp.zeros((npad,), jnp.float32)]
      ).reshape(NW, NCHUNK, CH),
      jnp.zeros((NW, 4, CH), jnp.float32)], axis=1)
  src2_p = jnp.concatenate([src_p, src_p + N_PAD], axis=0)

  z16 = jnp.zeros((N_PAD, 16), jnp.float32)
  z64 = jnp.zeros((N_PAD, 64), jnp.float32)
  z80 = jnp.zeros((N_PAD, 80), jnp.float32)
  z128 = jnp.zeros((N_PAD, 128), jnp.float32)
  ones16 = jnp.ones((N, 16), jnp.float32)

  _rep = lambda t: jnp.broadcast_to(
      t.reshape(E_TAB, 1), (E_TAB, L)).reshape(NW, NCHT, CH, L)
  ew_rep = _rep(ew_p)
  deg_h = _prop(16, 1)(ones16, src_p, dst_p, ew_rep, z16)
  recip = _tc_recip(deg_h)
  wT = _NORM(recip, dst_p, ew_p)
  w_rep = _rep(wT)

  b2 = lambda b: b.reshape(1, -1)
  Wi0 = Wi[0:H]
  Wo0, Wo1 = Wo[0:H], Wo[H:2 * H]
  Wro0, Wro1 = Wro[0:H], Wro[H:2 * H]
  wr = (Wr[0:IN], Wr[32:96], Wr[96:112], Wr[128:192], Wr[224:288], b2(br))
  wu = (Wu[0:IN], Wu[32:96], Wu[96:112], Wu[128:192], Wu[224:288], b2(bu))
  Wr2a, Wu2a = Wr[192:208], Wu[192:208]
  Wc0a, Wc0b, Wc1a, Wc2a = Wc[0:IN], Wc[32:96], Wc[96:112], Wc[192:208]
  Wc1b, Wc2b = Wc[128:192], Wc[224:288]
  alpha2 = alpha.reshape(1, 1)

  h = jnp.zeros((N, H), jnp.float32)
  gens, preds, reprs, states = [], [], [], []
  xs1, y1 = _tc_a(h, Wf, b2(bf), Wi0, b2(bi))
  for step in range(S):
    C1 = _prop(128, 1)(y1, src_p, dst_p, w_rep, z128)
    rep, xs2, y2 = _tc_b(C1, h, Wg, b2(bg), Wo0, Wo1, b2(bo), alpha2,
                         Wro0, Wro1, b2(bro))
    C2 = _prop(80, 1)(y2, src_p, dst_p, w_rep, z80)
    Axs2, prer, preu = _tc_c(C2, xs2, h, y2, wr, wu)
    C3 = _prop(16, 1)(Axs2, src_p, dst_p, w_rep, z16)
    rh, prec, u = _tc_d(C3, prer, preu, h, xs2, Axs2,
                        Wr2a, Wu2a, Wc0a, Wc0b, Wc1a, Wc2a, b2(bc))
    C4 = _prop(64, 1)(rh, src_p, dst_p, w_rep, z64)
    C5 = _prop(64, 2)(C4.reshape(2 * N_PAD, H), src2_p, dst_p, w_rep, z64)
    gens.append(xs2)
    preds.append(xs1)
    reprs.append(rep)
    if step < S - 1:
      h, xs1, y1 = _tc_fa(C4, C5, prec, u, h, Wc1b, Wc2b,
                          Wf, b2(bf), Wi0, b2(bi))
    else:
      h = _tc_f(C4, C5, prec, u, h, Wc1b, Wc2b)
    states.append(h)

  generations = jnp.stack(gens, 0)[None]
  predictions = jnp.stack(preds, 0)[None]
  representations = jnp.stack(reprs, 0)[None]
  states_out = jnp.stack(states, 0)[None, None]
  return generations, predictions, representations, states_out
